# Initial kernel scaffold; baseline (speedup 1.0000x reference)
import jax
import jax.numpy as jnp
from jax.experimental import pallas as pl


def kernel(x, edge_index, params):
    raise NotImplementedError("write your pallas kernel here")


# speedup vs baseline: 7.8177x; 7.8177x over previous
import functools

import jax
import jax.numpy as jnp
from jax import lax
from jax.experimental import pallas as pl
from jax.experimental.pallas import tpu as pltpu
from jax.experimental.pallas import tpu_sc as plsc

NN = 100000
NE = 1600000
NC, NS = 2, 16
NWK = NC * NS
CH = 128
GRP = 8
GE = CH * GRP
EPAD = NWK * 392 * CH
ROWS = EPAD // CH
NP = 100096
ACC_R = NP
WCH = 2048
F32 = jnp.float32

_SC_PARAMS = pltpu.CompilerParams(use_tc_tiling_on_sc=False)


def _sds(shape):
    return jax.ShapeDtypeStruct(shape, F32)


def _mesh():
    return plsc.VectorSubcoreMesh(core_axis_name="c", subcore_axis_name="s")


def _stripe_chunks(n):
    stripe = NP // 16
    out = [(i * n, n) for i in range(stripe // n)]
    if stripe % n:
        out.append((stripe // n * n, stripe % n))
    return out


def _stats(x128):
    def body(x_ref, o_ref):
        x = x_ref[...]
        s = jnp.sum(x, axis=0, keepdims=True)
        s2 = jnp.sum(x * x, axis=0, keepdims=True)
        o_ref[...] = jnp.concatenate(
            [s, s2, jnp.zeros((6, 128), F32)], axis=0)
    return pl.pallas_call(body, out_shape=_sds((8, 128)))(x128)


def _tables0(x4, stats, msel, bn2, bda, bdb, bias4):
    def body(x_ref, st_ref, ms_ref, bn_ref, a_ref, b_ref, c_ref,
             p_ref, q_ref):
        st = st_ref[...]
        ms = ms_ref[...]
        mean = jnp.dot(st[0:1], ms, preferred_element_type=F32,
                    precision=lax.Precision.HIGHEST) / NN
        ex2 = jnp.dot(st[1:2], ms, preferred_element_type=F32,
                    precision=lax.Precision.HIGHEST) / NN
        var = ex2 - mean * mean
        scale = bn_ref[0:1] / jnp.sqrt(var + 1e-5)
        shift = bn_ref[1:2] - mean * scale
        h = x_ref[...] * jnp.tile(scale, (1, 4)) + jnp.tile(shift, (1, 4))
        p_ref[...] = jnp.dot(h, a_ref[...], preferred_element_type=F32,
                    precision=lax.Precision.HIGHEST) \
            + c_ref[...]
        q_ref[...] = jnp.dot(h, b_ref[...], preferred_element_type=F32,
                    precision=lax.Precision.HIGHEST)
    return pl.pallas_call(
        body,
        grid=(5,),
        in_specs=[
            pl.BlockSpec((NN // 20, 16), lambda i: (i, 0)),
            pl.BlockSpec((8, 128), lambda i: (0, 0)),
            pl.BlockSpec((128, 4), lambda i: (0, 0)),
            pl.BlockSpec((2, 4), lambda i: (0, 0)),
            pl.BlockSpec((16, 128), lambda i: (0, 0)),
            pl.BlockSpec((16, 128), lambda i: (0, 0)),
            pl.BlockSpec((1, 128), lambda i: (0, 0)),
        ],
        out_specs=[pl.BlockSpec((NN // 20, 128), lambda i: (i, 0))] * 2,
        out_shape=[_sds((NN // 4, 128))] * 2,
    )(x4, stats, msel, bn2, bda, bdb, bias4)


def _cnt_expand(c0, c1, bc8, d0, d1):
    def body(c0_ref, c1_ref, bc_ref, d0_ref, d1_ref, i16_ref, i8_ref):
        inv = 1.0 / jnp.maximum(c0_ref[...] + c1_ref[...], 1.0)
        i16 = jnp.dot(inv, bc_ref[...], preferred_element_type=F32,
                    precision=lax.Precision.HIGHEST)
        i16_ref[...] = i16
        a8 = jnp.dot(i16, d0_ref[...], preferred_element_type=F32,
                    precision=lax.Precision.HIGHEST)
        b8 = jnp.dot(i16, d1_ref[...], preferred_element_type=F32,
                    precision=lax.Precision.HIGHEST)
        i8_ref[...] = jnp.concatenate([a8, b8], axis=1)
    return pl.pallas_call(
        body,
        grid=(2,),
        in_specs=[
            pl.BlockSpec((3128, 128), lambda i: (i, 0)),
            pl.BlockSpec((3128, 128), lambda i: (i, 0)),
            pl.BlockSpec((128, 128), lambda i: (0, 0)),
            pl.BlockSpec((128, 128), lambda i: (0, 0)),
            pl.BlockSpec((128, 128), lambda i: (0, 0)),
        ],
        out_specs=[pl.BlockSpec((3128, 128), lambda i: (i, 0)),
                   pl.BlockSpec((3128, 256), lambda i: (i, 0))],
        out_shape=[_sds((NP // 16, 128)), _sds((NP // 16, 256))],
    )(c0, c1, bc8, d0, d1)


def _tables_wide(slo8, shi8, inv8, ma, mb, bias8):
    def body(lo_ref, hi_ref, iv_ref, a_ref, b_ref, c_ref, p_ref, q_ref):
        iv = iv_ref[...]
        h = jnp.concatenate([lo_ref[...] * iv, hi_ref[...] * iv], axis=1)
        p_ref[...] = jnp.dot(h, a_ref[...], preferred_element_type=F32,
                    precision=lax.Precision.HIGHEST) \
            + c_ref[...]
        q_ref[...] = jnp.dot(h, b_ref[...], preferred_element_type=F32,
                    precision=lax.Precision.HIGHEST)
    return pl.pallas_call(
        body,
        grid=(4,),
        in_specs=[
            pl.BlockSpec((3128, 128), lambda i: (i, 0)),
            pl.BlockSpec((3128, 128), lambda i: (i, 0)),
            pl.BlockSpec((3128, 128), lambda i: (i, 0)),
            pl.BlockSpec((256, 256), lambda i: (0, 0)),
            pl.BlockSpec((256, 256), lambda i: (0, 0)),
            pl.BlockSpec((1, 256), lambda i: (0, 0)),
        ],
        out_specs=[pl.BlockSpec((3128, 256), lambda i: (i, 0))] * 2,
        out_shape=[_sds((NP // 8, 256))] * 2,
    )(slo8, shi8, inv8, ma, mb, bias8)


def _tables_narrow(s0, s1, inv16, mc, md, bias16):
    def body(s0_ref, s1_ref, iv_ref, a_ref, b_ref, c_ref, p_ref, q_ref):
        h = (s0_ref[...] + s1_ref[...]) * iv_ref[...]
        p_ref[...] = jnp.dot(h, a_ref[...], preferred_element_type=F32,
                    precision=lax.Precision.HIGHEST) \
            + c_ref[...]
        q_ref[...] = jnp.dot(h, b_ref[...], preferred_element_type=F32,
                    precision=lax.Precision.HIGHEST)
    return pl.pallas_call(
        body,
        grid=(2,),
        in_specs=[
            pl.BlockSpec((3128, 128), lambda i: (i, 0)),
            pl.BlockSpec((3128, 128), lambda i: (i, 0)),
            pl.BlockSpec((3128, 128), lambda i: (i, 0)),
            pl.BlockSpec((128, 512), lambda i: (0, 0)),
            pl.BlockSpec((128, 512), lambda i: (0, 0)),
            pl.BlockSpec((1, 512), lambda i: (0, 0)),
        ],
        out_specs=[pl.BlockSpec((3128, 512), lambda i: (i, 0))] * 2,
        out_shape=[_sds((NP // 16, 512))] * 2,
    )(s0, s1, inv16, mc, md, bias16)


def _mlp(last_relu, g1, g2, bdw2, b2t, bdw3, b3t):
    def body(g1_ref, g2_ref, w2_ref, b2_ref, w3_ref, b3_ref, m_ref):
        h1 = jnp.maximum(g1_ref[...] + g2_ref[...], 0.0)
        h2 = jnp.maximum(
            jnp.dot(h1, w2_ref[...], preferred_element_type=F32,
                    precision=lax.Precision.HIGHEST)
            + b2_ref[...], 0.0)
        m = jnp.dot(h2, w3_ref[...], preferred_element_type=F32,
                    precision=lax.Precision.HIGHEST) \
            + b3_ref[...]
        if last_relu:
            m = jnp.maximum(m, 0.0)
        m_ref[...] = m
    return pl.pallas_call(
        body,
        grid=(392,),
        in_specs=[
            pl.BlockSpec((1024, 128), lambda i: (i, 0)),
            pl.BlockSpec((1024, 128), lambda i: (i, 0)),
            pl.BlockSpec((128, 128), lambda i: (0, 0)),
            pl.BlockSpec((1, 128), lambda i: (0, 0)),
            pl.BlockSpec((128, 128), lambda i: (0, 0)),
            pl.BlockSpec((1, 128), lambda i: (0, 0)),
        ],
        out_specs=pl.BlockSpec((1024, 128), lambda i: (i, 0)),
        out_shape=_sds((EPAD // 4, 128)),
    )(g1, g2, bdw2, b2t, bdw3, b3t)


def _final(s0, s1, inv16, sel):
    def body(s0_ref, s1_ref, iv_ref, sel_ref, o_ref):
        v = (s0_ref[...] + s1_ref[...]) * iv_ref[...]
        o_ref[...] = jnp.dot(v, sel_ref[...], preferred_element_type=F32,
                    precision=lax.Precision.HIGHEST)
    return pl.pallas_call(
        body,
        grid=(2,),
        in_specs=[
            pl.BlockSpec((3128, 128), lambda i: (i, 0)),
            pl.BlockSpec((3128, 128), lambda i: (i, 0)),
            pl.BlockSpec((3128, 128), lambda i: (i, 0)),
            pl.BlockSpec((128, 64), lambda i: (0, 0)),
        ],
        out_specs=pl.BlockSpec((3128, 64), lambda i: (i, 0)),
        out_shape=_sds((NP // 16, 64)),
    )(s0, s1, inv16, sel)


def _sc_gather(ptab, qtab, dstg, srcg, dep):
    @functools.partial(
        pl.kernel,
        out_type=[_sds((EPAD, 32))] * 2,
        mesh=_mesh(),
        scratch_types=[
            pltpu.VMEM((GRP, CH), jnp.int32),
            pltpu.VMEM((GE, 32), F32),
            pltpu.SemaphoreType.DMA,
        ],
        compiler_params=_SC_PARAMS,
    )
    def k(p_hbm, q_hbm, dg_hbm, sg_hbm, dep_hbm, g1_hbm, g2_hbm, idx_v,
          rows_v, sem):
        wid = lax.axis_index("s") * NC + lax.axis_index("c")
        row0 = wid * (ROWS // NWK)

        def grp(g, carry):
            r = row0 + g * GRP
            base = r * CH
            pltpu.sync_copy(dg_hbm.at[pl.ds(r, GRP)], idx_v)
            cps = [pltpu.async_copy(p_hbm.at[idx_v.at[j]],
                                    rows_v.at[pl.ds(j * CH, CH)], sem)
                   for j in range(GRP)]
            for c in cps:
                c.wait()
            pltpu.sync_copy(rows_v, g1_hbm.at[pl.ds(base, GE)])
            pltpu.sync_copy(sg_hbm.at[pl.ds(r, GRP)], idx_v)
            cps = [pltpu.async_copy(q_hbm.at[idx_v.at[j]],
                                    rows_v.at[pl.ds(j * CH, CH)], sem)
                   for j in range(GRP)]
            for c in cps:
                c.wait()
            pltpu.sync_copy(rows_v, g2_hbm.at[pl.ds(base, GE)])
            return carry

        lax.fori_loop(0, (ROWS // NWK) // GRP, grp, 0)

    return k(ptab, qtab, dstg, srcg, dep)


def _sc_scatter_wide(m, dsts, zeros16):
    @functools.partial(
        pl.kernel,
        out_type=[_sds((NP, 16))] * 2,
        mesh=_mesh(),
        scratch_types=[
            pltpu.VMEM((GRP, CH), jnp.int32),
            pltpu.VMEM((GE, 16), F32),
            pltpu.VMEM((512, 16), F32),
            pltpu.VMEM_SHARED((ACC_R, 16), F32),
        ],
        compiler_params=_SC_PARAMS,
    )
    def k(m_hbm, ds_hbm, zr_hbm, slo_hbm, shi_hbm,
          idx_v, vals_v, zw_v, acc):
        cid = lax.axis_index("c")
        sid = lax.axis_index("s")
        pltpu.sync_copy(zr_hbm, zw_v)
        z0 = sid * (NP // 16)
        for off, sz in _stripe_chunks(512):
            pltpu.sync_copy(zw_v.at[pl.ds(0, sz)],
                            acc.at[pl.ds(z0 + off, sz)])
        plsc.subcore_barrier()

        row0 = sid * (ROWS // NS)

        def grp(g, carry):
            r = row0 + g * GRP
            pltpu.sync_copy(ds_hbm.at[pl.ds(r, GRP)], idx_v)

            @pl.when(cid == 0)
            def _():
                pltpu.sync_copy(
                    m_hbm.at[pl.ds(r * CH, GE), pl.ds(0, 16)], vals_v)

            @pl.when(cid == 1)
            def _():
                pltpu.sync_copy(
                    m_hbm.at[pl.ds(r * CH, GE), pl.ds(16, 16)], vals_v)

            for j in range(GRP):
                pltpu.sync_copy(vals_v.at[pl.ds(j * CH, CH)],
                                acc.at[idx_v.at[j]], add=True)
            return carry

        lax.fori_loop(0, (ROWS // NS) // GRP, grp, 0)
        plsc.subcore_barrier()

        w0 = sid * (NP // 16)
        for off, sz in _stripe_chunks(512):
            pltpu.sync_copy(acc.at[pl.ds(w0 + off, sz)],
                            zw_v.at[pl.ds(0, sz)])

            @pl.when(cid == 0)
            def _():
                pltpu.sync_copy(zw_v.at[pl.ds(0, sz)],
                                slo_hbm.at[pl.ds(w0 + off, sz)])

            @pl.when(cid == 1)
            def _():
                pltpu.sync_copy(zw_v.at[pl.ds(0, sz)],
                                shi_hbm.at[pl.ds(w0 + off, sz)])

    return k(m, dsts, zeros16)


def _sc_scatter_narrow(m, dsts, zeros8):
    @functools.partial(
        pl.kernel,
        out_type=_sds((2 * NP, 8)),
        mesh=_mesh(),
        scratch_types=[
            pltpu.VMEM((GRP, CH), jnp.int32),
            pltpu.VMEM((GE, 8), F32),
            pltpu.VMEM((WCH, 8), F32),
            pltpu.VMEM_SHARED((ACC_R, 8), F32),
        ],
        compiler_params=_SC_PARAMS,
    )
    def k(m_hbm, ds_hbm, zr_hbm, out_hbm, idx_v, vals_v, zw_v, acc):
        cid = lax.axis_index("c")
        sid = lax.axis_index("s")
        pltpu.sync_copy(zr_hbm, zw_v)
        z0 = sid * (NP // 16)
        for off, sz in _stripe_chunks(WCH):
            pltpu.sync_copy(zw_v.at[pl.ds(0, sz)],
                            acc.at[pl.ds(z0 + off, sz)])
        plsc.subcore_barrier()

        row0 = cid * (ROWS // NC) + sid * (ROWS // NWK)

        def grp(g, carry):
            r = row0 + g * GRP
            pltpu.sync_copy(ds_hbm.at[pl.ds(r, GRP)], idx_v)
            pltpu.sync_copy(m_hbm.at[pl.ds(r * CH, GE), pl.ds(0, 8)],
                            vals_v)
            for j in range(GRP):
                pltpu.sync_copy(vals_v.at[pl.ds(j * CH, CH)],
                                acc.at[idx_v.at[j]], add=True)
            return carry

        lax.fori_loop(0, (ROWS // NWK) // GRP, grp, 0)
        plsc.subcore_barrier()

        w0 = sid * (NP // 16)
        for off, sz in _stripe_chunks(WCH):
            pltpu.sync_copy(acc.at[pl.ds(w0 + off, sz)],
                            zw_v.at[pl.ds(0, sz)])
            pltpu.sync_copy(zw_v.at[pl.ds(0, sz)],
                            out_hbm.at[pl.ds(cid * NP + w0 + off, sz)])

    return k(m, dsts, zeros8)


def _sc_count(dsts, cvals, zeros8):
    @functools.partial(
        pl.kernel,
        out_type=_sds((2 * NP, 8)),
        mesh=_mesh(),
        scratch_types=[
            pltpu.VMEM((GRP, CH), jnp.int32),
            pltpu.VMEM((CH, 8), F32),
            pltpu.VMEM((WCH, 8), F32),
            pltpu.VMEM_SHARED((ACC_R, 8), F32),
        ],
        compiler_params=_SC_PARAMS,
    )
    def k(ds_hbm, cv_hbm, zr_hbm, out_hbm, idx_v, vals_v, zw_v, acc):
        cid = lax.axis_index("c")
        sid = lax.axis_index("s")
        pltpu.sync_copy(zr_hbm, zw_v)
        z0 = sid * (NP // 16)
        for off, sz in _stripe_chunks(WCH):
            pltpu.sync_copy(zw_v.at[pl.ds(0, sz)],
                            acc.at[pl.ds(z0 + off, sz)])
        pltpu.sync_copy(cv_hbm, vals_v)
        plsc.subcore_barrier()

        row0 = cid * (ROWS // NC) + sid * (ROWS // NWK)

        def grp(g, carry):
            r = row0 + g * GRP
            pltpu.sync_copy(ds_hbm.at[pl.ds(r, GRP)], idx_v)
            for j in range(GRP):
                pltpu.sync_copy(vals_v, acc.at[idx_v.at[j]], add=True)
            return carry

        lax.fori_loop(0, (ROWS // NWK) // GRP, grp, 0)
        plsc.subcore_barrier()

        w0 = sid * (NP // 16)
        for off, sz in _stripe_chunks(WCH):
            pltpu.sync_copy(acc.at[pl.ds(w0 + off, sz)],
                            zw_v.at[pl.ds(0, sz)])
            pltpu.sync_copy(zw_v.at[pl.ds(0, sz)],
                            out_hbm.at[pl.ds(cid * NP + w0 + off, sz)])

    return k(dsts, cvals, zeros8)


def _prep_conv(p, fdim, fout):
    w0, w1, w2 = p["W"]
    b0, b1, b2 = p["b"]
    wa = w0[:fdim] - w0[fdim:]
    wb = w0[fdim:]
    eye4 = jnp.eye(4, dtype=F32)
    if fout < 32:
        w2 = jnp.concatenate([w2, jnp.zeros((32, 32 - fout), F32)],
                             axis=1)
        b2 = jnp.concatenate([b2, jnp.zeros((32 - fout,), F32)])
    d = {
        "bdw2": jnp.kron(eye4, w1),
        "b2t": jnp.tile(b1.reshape(1, 32), (1, 4)),
        "bdw3": jnp.kron(eye4, w2),
        "b3t": jnp.tile(b2.reshape(1, 32), (1, 4)),
    }
    if fdim == 4:
        d["ta"] = jnp.kron(eye4, wa)
        d["tb"] = jnp.kron(eye4, wb)
        d["tbias"] = jnp.tile(b0.reshape(1, 32), (1, 4))
    elif fdim == 32:
        eye8 = jnp.eye(8, dtype=F32)
        d["ta"] = jnp.concatenate(
            [jnp.kron(eye8, wa[:16]), jnp.kron(eye8, wa[16:])], axis=0)
        d["tb"] = jnp.concatenate(
            [jnp.kron(eye8, wb[:16]), jnp.kron(eye8, wb[16:])], axis=0)
        d["tbias"] = jnp.tile(b0.reshape(1, 32), (1, 8))
    else:
        eye16 = jnp.eye(16, dtype=F32)
        wap = jnp.concatenate([wa, jnp.zeros((6, 32), F32)], axis=0)
        wbp = jnp.concatenate([wb, jnp.zeros((6, 32), F32)], axis=0)
        d["ta"] = jnp.kron(eye16, wap)
        d["tb"] = jnp.kron(eye16, wbp)
        d["tbias"] = jnp.tile(b0.reshape(1, 32), (1, 16))
    return d


def kernel(x, edge_index, params):
    src = edge_index[0].astype(jnp.int32)
    dst = edge_index[1].astype(jnp.int32)
    npad = EPAD - NE
    pad_g = (jnp.arange(npad, dtype=jnp.int32) * 97) % NN
    pad_s = NN + (jnp.arange(npad, dtype=jnp.int32) % 8)
    dstg = jnp.concatenate([dst, pad_g]).reshape(ROWS, CH)
    srcg = jnp.concatenate([src, pad_g]).reshape(ROWS, CH)
    dsts = jnp.concatenate([dst, pad_s]).reshape(ROWS, CH)

    x128 = x.reshape(NN // 32, 128)
    x4 = x.reshape(NN // 4, 16)
    msel = (jnp.arange(128, dtype=jnp.int32)[:, None] % 4
            == jnp.arange(4, dtype=jnp.int32)[None, :]).astype(F32)
    bn2 = jnp.stack([params["bn"]["gamma"], params["bn"]["beta"]])
    zeros16 = jnp.zeros((512, 16), F32)
    zeros8 = jnp.zeros((WCH, 8), F32)
    cvals = (jnp.arange(8, dtype=jnp.int32)[None, :] == 0
             ).astype(F32) * jnp.ones((CH, 1), F32)

    lanes = jnp.arange(128)
    bc8 = jnp.kron(jnp.eye(16, dtype=F32),
                   jnp.zeros((8, 8), F32).at[0].set(1.0))
    d0 = jnp.zeros((128, 128), F32).at[(lanes // 16) * 8, lanes].set(1.0)
    d1 = jnp.zeros((128, 128), F32).at[64 + (lanes // 16) * 8,
                                       lanes].set(1.0)
    l64 = jnp.arange(64)
    sel = jnp.zeros((128, 64), F32).at[(l64 // 4) * 8 + l64 % 4,
                                       l64].set(1.0)

    e1 = _prep_conv(params["enc1"], 4, 32)
    e2 = _prep_conv(params["enc2"], 32, 2)
    dc1 = _prep_conv(params["dec1"], 2, 32)
    dc2 = _prep_conv(params["dec2"], 32, 4)

    cnt2 = _sc_count(dsts, cvals, zeros8)
    dep = cnt2[:8]
    c0 = cnt2[:NP].reshape(NP // 16, 128)
    c1 = cnt2[NP:].reshape(NP // 16, 128)
    inv16, inv8w = _cnt_expand(c0, c1, bc8, d0, d1)
    inv8 = inv8w.reshape(NP // 8, 128)

    stats = _stats(x128)
    p, q = _tables0(x4, stats, msel, bn2, e1["ta"], e1["tb"], e1["tbias"])

    def as_tab(t):
        return t.reshape(-1, 32)

    def g128(g):
        return g.reshape(EPAD // 4, 128)

    g1, g2 = _sc_gather(as_tab(p), as_tab(q), dstg, srcg, dep)
    m = _mlp(True, g128(g1), g128(g2), e1["bdw2"], e1["b2t"],
             e1["bdw3"], e1["b3t"])
    slo, shi = _sc_scatter_wide(m.reshape(EPAD, 32), dsts, zeros16)
    p, q = _tables_wide(slo.reshape(NP // 8, 128),
                        shi.reshape(NP // 8, 128), inv8,
                        e2["ta"], e2["tb"], e2["tbias"])
    g1, g2 = _sc_gather(as_tab(p), as_tab(q), dstg, srcg, dep)
    m = _mlp(True, g128(g1), g128(g2), e2["bdw2"], e2["b2t"],
             e2["bdw3"], e2["b3t"])
    s8 = _sc_scatter_narrow(m.reshape(EPAD, 32), dsts, zeros8)
    p, q = _tables_narrow(s8[:NP].reshape(NP // 16, 128),
                          s8[NP:].reshape(NP // 16, 128), inv16,
                          dc1["ta"], dc1["tb"], dc1["tbias"])
    g1, g2 = _sc_gather(as_tab(p), as_tab(q), dstg, srcg, dep)
    m = _mlp(True, g128(g1), g128(g2), dc1["bdw2"], dc1["b2t"],
             dc1["bdw3"], dc1["b3t"])
    slo, shi = _sc_scatter_wide(m.reshape(EPAD, 32), dsts, zeros16)
    p, q = _tables_wide(slo.reshape(NP // 8, 128),
                        shi.reshape(NP // 8, 128), inv8,
                        dc2["ta"], dc2["tb"], dc2["tbias"])
    g1, g2 = _sc_gather(as_tab(p), as_tab(q), dstg, srcg, dep)
    m = _mlp(False, g128(g1), g128(g2), dc2["bdw2"], dc2["b2t"],
             dc2["bdw3"], dc2["b3t"])
    s8 = _sc_scatter_narrow(m.reshape(EPAD, 32), dsts, zeros8)

    out = _final(s8[:NP].reshape(NP // 16, 128),
                 s8[NP:].reshape(NP // 16, 128), inv16, sel)
    return out.reshape(NP, 4)[:NN]


# speedup vs baseline: 8.8218x; 1.1284x over previous
import functools

import jax
import jax.numpy as jnp
from jax import lax
from jax.experimental import pallas as pl
from jax.experimental.pallas import tpu as pltpu
from jax.experimental.pallas import tpu_sc as plsc

NN = 100000
NE = 1600000
NC, NS = 2, 16
NWK = NC * NS
CH = 128
GRP = 8
GE = CH * GRP
EPAD = NWK * 392 * CH
ROWS = EPAD // CH
NP = 100096
ACC_R = NP
WCH = 2048
F32 = jnp.float32

_SC_PARAMS = pltpu.CompilerParams(use_tc_tiling_on_sc=False)


def _sds(shape):
    return jax.ShapeDtypeStruct(shape, F32)


def _mesh():
    return plsc.VectorSubcoreMesh(core_axis_name="c", subcore_axis_name="s")


def _stripe_chunks(n):
    stripe = NP // 16
    out = [(i * n, n) for i in range(stripe // n)]
    if stripe % n:
        out.append((stripe // n * n, stripe % n))
    return out


def _stats(x128):
    def body(x_ref, o_ref):
        x = x_ref[...]
        s = jnp.sum(x, axis=0, keepdims=True)
        s2 = jnp.sum(x * x, axis=0, keepdims=True)
        o_ref[...] = jnp.concatenate(
            [s, s2, jnp.zeros((6, 128), F32)], axis=0)
    return pl.pallas_call(body, out_shape=_sds((8, 128)))(x128)


def _tables0(x4, stats, msel, bn2, bda, bdb, bias4):
    def body(x_ref, st_ref, ms_ref, bn_ref, a_ref, b_ref, c_ref,
             p_ref, q_ref):
        st = st_ref[...]
        ms = ms_ref[...]
        mean = jnp.dot(st[0:1], ms, preferred_element_type=F32,
                    precision=lax.Precision.HIGHEST) / NN
        ex2 = jnp.dot(st[1:2], ms, preferred_element_type=F32,
                    precision=lax.Precision.HIGHEST) / NN
        var = ex2 - mean * mean
        scale = bn_ref[0:1] / jnp.sqrt(var + 1e-5)
        shift = bn_ref[1:2] - mean * scale
        h = x_ref[...] * jnp.tile(scale, (1, 4)) + jnp.tile(shift, (1, 4))
        p_ref[...] = jnp.dot(h, a_ref[...], preferred_element_type=F32,
                    precision=lax.Precision.HIGHEST) \
            + c_ref[...]
        q_ref[...] = jnp.dot(h, b_ref[...], preferred_element_type=F32,
                    precision=lax.Precision.HIGHEST)
    return pl.pallas_call(
        body,
        grid=(5,),
        in_specs=[
            pl.BlockSpec((NN // 20, 16), lambda i: (i, 0)),
            pl.BlockSpec((8, 128), lambda i: (0, 0)),
            pl.BlockSpec((128, 4), lambda i: (0, 0)),
            pl.BlockSpec((2, 4), lambda i: (0, 0)),
            pl.BlockSpec((16, 128), lambda i: (0, 0)),
            pl.BlockSpec((16, 128), lambda i: (0, 0)),
            pl.BlockSpec((1, 128), lambda i: (0, 0)),
        ],
        out_specs=[pl.BlockSpec((NN // 20, 128), lambda i: (i, 0))] * 2,
        out_shape=[_sds((NN // 4, 128))] * 2,
    )(x4, stats, msel, bn2, bda, bdb, bias4)


def _cnt_expand(c0, c1, bc8, d0, d1):
    def body(c0_ref, c1_ref, bc_ref, d0_ref, d1_ref, i16_ref, i8_ref):
        inv = 1.0 / jnp.maximum(c0_ref[...] + c1_ref[...], 1.0)
        i16 = jnp.dot(inv, bc_ref[...], preferred_element_type=F32,
                    precision=lax.Precision.HIGHEST)
        i16_ref[...] = i16
        a8 = jnp.dot(i16, d0_ref[...], preferred_element_type=F32,
                    precision=lax.Precision.HIGHEST)
        b8 = jnp.dot(i16, d1_ref[...], preferred_element_type=F32,
                    precision=lax.Precision.HIGHEST)
        i8_ref[...] = jnp.concatenate([a8, b8], axis=1)
    return pl.pallas_call(
        body,
        grid=(2,),
        in_specs=[
            pl.BlockSpec((3128, 128), lambda i: (i, 0)),
            pl.BlockSpec((3128, 128), lambda i: (i, 0)),
            pl.BlockSpec((128, 128), lambda i: (0, 0)),
            pl.BlockSpec((128, 128), lambda i: (0, 0)),
            pl.BlockSpec((128, 128), lambda i: (0, 0)),
        ],
        out_specs=[pl.BlockSpec((3128, 128), lambda i: (i, 0)),
                   pl.BlockSpec((3128, 256), lambda i: (i, 0))],
        out_shape=[_sds((NP // 16, 128)), _sds((NP // 16, 256))],
    )(c0, c1, bc8, d0, d1)


def _tables_wide(slo8, shi8, inv8, ma, mb, bias8):
    def body(lo_ref, hi_ref, iv_ref, a_ref, b_ref, c_ref, p_ref, q_ref):
        iv = iv_ref[...]
        h = jnp.concatenate([lo_ref[...] * iv, hi_ref[...] * iv], axis=1)
        p_ref[...] = jnp.dot(h, a_ref[...], preferred_element_type=F32,
                    precision=lax.Precision.HIGHEST) \
            + c_ref[...]
        q_ref[...] = jnp.dot(h, b_ref[...], preferred_element_type=F32,
                    precision=lax.Precision.HIGHEST)
    return pl.pallas_call(
        body,
        grid=(4,),
        in_specs=[
            pl.BlockSpec((3128, 128), lambda i: (i, 0)),
            pl.BlockSpec((3128, 128), lambda i: (i, 0)),
            pl.BlockSpec((3128, 128), lambda i: (i, 0)),
            pl.BlockSpec((256, 256), lambda i: (0, 0)),
            pl.BlockSpec((256, 256), lambda i: (0, 0)),
            pl.BlockSpec((1, 256), lambda i: (0, 0)),
        ],
        out_specs=[pl.BlockSpec((3128, 256), lambda i: (i, 0))] * 2,
        out_shape=[_sds((NP // 8, 256))] * 2,
    )(slo8, shi8, inv8, ma, mb, bias8)


def _tables_narrow(s0, s1, inv16, mc, md, bias16):
    def body(s0_ref, s1_ref, iv_ref, a_ref, b_ref, c_ref, p_ref, q_ref):
        h = (s0_ref[...] + s1_ref[...]) * iv_ref[...]
        p_ref[...] = jnp.dot(h, a_ref[...], preferred_element_type=F32,
                    precision=lax.Precision.HIGHEST) \
            + c_ref[...]
        q_ref[...] = jnp.dot(h, b_ref[...], preferred_element_type=F32,
                    precision=lax.Precision.HIGHEST)
    return pl.pallas_call(
        body,
        grid=(2,),
        in_specs=[
            pl.BlockSpec((3128, 128), lambda i: (i, 0)),
            pl.BlockSpec((3128, 128), lambda i: (i, 0)),
            pl.BlockSpec((3128, 128), lambda i: (i, 0)),
            pl.BlockSpec((128, 512), lambda i: (0, 0)),
            pl.BlockSpec((128, 512), lambda i: (0, 0)),
            pl.BlockSpec((1, 512), lambda i: (0, 0)),
        ],
        out_specs=[pl.BlockSpec((3128, 512), lambda i: (i, 0))] * 2,
        out_shape=[_sds((NP // 16, 512))] * 2,
    )(s0, s1, inv16, mc, md, bias16)


def _mlp(last_relu, g1, g2, bdw2, b2t, bdw3, b3t):
    def body(g1_ref, g2_ref, w2_ref, b2_ref, w3_ref, b3_ref, m_ref):
        h1 = jnp.maximum(g1_ref[...] + g2_ref[...], 0.0)
        h2 = jnp.maximum(
            jnp.dot(h1, w2_ref[...], preferred_element_type=F32,
                    precision=lax.Precision.HIGHEST)
            + b2_ref[...], 0.0)
        m = jnp.dot(h2, w3_ref[...], preferred_element_type=F32,
                    precision=lax.Precision.HIGHEST) \
            + b3_ref[...]
        if last_relu:
            m = jnp.maximum(m, 0.0)
        m_ref[...] = m
    return pl.pallas_call(
        body,
        grid=(392,),
        in_specs=[
            pl.BlockSpec((1024, 128), lambda i: (i, 0)),
            pl.BlockSpec((1024, 128), lambda i: (i, 0)),
            pl.BlockSpec((128, 128), lambda i: (0, 0)),
            pl.BlockSpec((1, 128), lambda i: (0, 0)),
            pl.BlockSpec((128, 128), lambda i: (0, 0)),
            pl.BlockSpec((1, 128), lambda i: (0, 0)),
        ],
        out_specs=pl.BlockSpec((1024, 128), lambda i: (i, 0)),
        out_shape=_sds((EPAD // 4, 128)),
    )(g1, g2, bdw2, b2t, bdw3, b3t)


def _final(s0, s1, inv16, sel):
    def body(s0_ref, s1_ref, iv_ref, sel_ref, o_ref):
        v = (s0_ref[...] + s1_ref[...]) * iv_ref[...]
        o_ref[...] = jnp.dot(v, sel_ref[...], preferred_element_type=F32,
                    precision=lax.Precision.HIGHEST)
    return pl.pallas_call(
        body,
        grid=(2,),
        in_specs=[
            pl.BlockSpec((3128, 128), lambda i: (i, 0)),
            pl.BlockSpec((3128, 128), lambda i: (i, 0)),
            pl.BlockSpec((3128, 128), lambda i: (i, 0)),
            pl.BlockSpec((128, 64), lambda i: (0, 0)),
        ],
        out_specs=pl.BlockSpec((3128, 64), lambda i: (i, 0)),
        out_shape=_sds((NP // 16, 64)),
    )(s0, s1, inv16, sel)


def _sc_gather(ptab, qtab, dstg, srcg, dep):
    @functools.partial(
        pl.kernel,
        out_type=[_sds((EPAD, 32))] * 2,
        mesh=_mesh(),
        scratch_types=[
            pltpu.VMEM((GRP, CH), jnp.int32),
            pltpu.VMEM((GRP, CH), jnp.int32),
            pltpu.VMEM((GE, 32), F32),
            pltpu.VMEM((GE, 32), F32),
            pltpu.SemaphoreType.DMA,
            pltpu.SemaphoreType.DMA,
            pltpu.SemaphoreType.DMA,
        ],
        compiler_params=_SC_PARAMS,
    )
    def k(p_hbm, q_hbm, dg_hbm, sg_hbm, dep_hbm, g1_hbm, g2_hbm,
          idxd_v, idxs_v, rowsp_v, rowsq_v, seml, semg, semw):
        wid = lax.axis_index("s") * NC + lax.axis_index("c")
        row0 = wid * (ROWS // NWK)

        def grp(g, carry):
            r = row0 + g * GRP
            base = r * CH
            ci = pltpu.async_copy(dg_hbm.at[pl.ds(r, GRP)], idxd_v, seml)
            cs = pltpu.async_copy(sg_hbm.at[pl.ds(r, GRP)], idxs_v, seml)
            ci.wait()
            cps_p = [pltpu.async_copy(p_hbm.at[idxd_v.at[j]],
                                      rowsp_v.at[pl.ds(j * CH, CH)],
                                      semg)
                     for j in range(GRP)]
            cs.wait()
            cps_q = [pltpu.async_copy(q_hbm.at[idxs_v.at[j]],
                                      rowsq_v.at[pl.ds(j * CH, CH)],
                                      semg)
                     for j in range(GRP)]
            for c in cps_p:
                c.wait()
            w1 = pltpu.async_copy(rowsp_v, g1_hbm.at[pl.ds(base, GE)],
                                  semw)
            for c in cps_q:
                c.wait()
            w2 = pltpu.async_copy(rowsq_v, g2_hbm.at[pl.ds(base, GE)],
                                  semw)
            w1.wait()
            w2.wait()
            return carry

        lax.fori_loop(0, (ROWS // NWK) // GRP, grp, 0)

    return k(ptab, qtab, dstg, srcg, dep)


def _sc_scatter_wide(m, dsts, zeros16):
    @functools.partial(
        pl.kernel,
        out_type=[_sds((NP, 16))] * 2,
        mesh=_mesh(),
        scratch_types=[
            pltpu.VMEM((GRP, CH), jnp.int32),
            pltpu.VMEM((GE, 16), F32),
            pltpu.VMEM((512, 16), F32),
            pltpu.VMEM_SHARED((ACC_R, 16), F32),
            pltpu.SemaphoreType.DMA,
            pltpu.SemaphoreType.DMA,
        ],
        compiler_params=_SC_PARAMS,
    )
    def k(m_hbm, ds_hbm, zr_hbm, slo_hbm, shi_hbm,
          idx_v, vals_v, zw_v, acc, seml, sems):
        cid = lax.axis_index("c")
        sid = lax.axis_index("s")
        pltpu.sync_copy(zr_hbm, zw_v)
        z0 = sid * (NP // 16)
        for off, sz in _stripe_chunks(512):
            pltpu.sync_copy(zw_v.at[pl.ds(0, sz)],
                            acc.at[pl.ds(z0 + off, sz)])
        plsc.subcore_barrier()

        row0 = sid * (ROWS // NS)

        def grp(g, carry):
            r = row0 + g * GRP
            ci = pltpu.async_copy(ds_hbm.at[pl.ds(r, GRP)], idx_v, seml)

            @pl.when(cid == 0)
            def _():
                pltpu.async_copy(
                    m_hbm.at[pl.ds(r * CH, GE), pl.ds(0, 16)], vals_v,
                    seml)

            @pl.when(cid == 1)
            def _():
                pltpu.async_copy(
                    m_hbm.at[pl.ds(r * CH, GE), pl.ds(16, 16)], vals_v,
                    seml)

            ci.wait()
            cv = pltpu.make_async_copy(
                m_hbm.at[pl.ds(r * CH, GE), pl.ds(0, 16)], vals_v, seml)
            cv.wait()
            cps = [pltpu.async_copy(vals_v.at[pl.ds(j * CH, CH)],
                                    acc.at[idx_v.at[j]], sems, add=True)
                   for j in range(GRP)]
            for c in cps:
                c.wait()
            return carry

        lax.fori_loop(0, (ROWS // NS) // GRP, grp, 0)
        plsc.subcore_barrier()

        w0 = sid * (NP // 16)
        for off, sz in _stripe_chunks(512):
            pltpu.sync_copy(acc.at[pl.ds(w0 + off, sz)],
                            zw_v.at[pl.ds(0, sz)])

            @pl.when(cid == 0)
            def _():
                pltpu.sync_copy(zw_v.at[pl.ds(0, sz)],
                                slo_hbm.at[pl.ds(w0 + off, sz)])

            @pl.when(cid == 1)
            def _():
                pltpu.sync_copy(zw_v.at[pl.ds(0, sz)],
                                shi_hbm.at[pl.ds(w0 + off, sz)])

    return k(m, dsts, zeros16)


def _sc_scatter_narrow(m, dsts, zeros8):
    @functools.partial(
        pl.kernel,
        out_type=_sds((2 * NP, 8)),
        mesh=_mesh(),
        scratch_types=[
            pltpu.VMEM((GRP, CH), jnp.int32),
            pltpu.VMEM((GE, 8), F32),
            pltpu.VMEM((WCH, 8), F32),
            pltpu.VMEM_SHARED((ACC_R, 8), F32),
            pltpu.SemaphoreType.DMA,
            pltpu.SemaphoreType.DMA,
        ],
        compiler_params=_SC_PARAMS,
    )
    def k(m_hbm, ds_hbm, zr_hbm, out_hbm, idx_v, vals_v, zw_v, acc,
          seml, sems):
        cid = lax.axis_index("c")
        sid = lax.axis_index("s")
        pltpu.sync_copy(zr_hbm, zw_v)
        z0 = sid * (NP // 16)
        for off, sz in _stripe_chunks(WCH):
            pltpu.sync_copy(zw_v.at[pl.ds(0, sz)],
                            acc.at[pl.ds(z0 + off, sz)])
        plsc.subcore_barrier()

        row0 = cid * (ROWS // NC) + sid * (ROWS // NWK)

        def grp(g, carry):
            r = row0 + g * GRP
            ci = pltpu.async_copy(ds_hbm.at[pl.ds(r, GRP)], idx_v, seml)
            cv = pltpu.async_copy(
                m_hbm.at[pl.ds(r * CH, GE), pl.ds(0, 8)], vals_v, seml)
            ci.wait()
            cv.wait()
            cps = [pltpu.async_copy(vals_v.at[pl.ds(j * CH, CH)],
                                    acc.at[idx_v.at[j]], sems, add=True)
                   for j in range(GRP)]
            for c in cps:
                c.wait()
            return carry

        lax.fori_loop(0, (ROWS // NWK) // GRP, grp, 0)
        plsc.subcore_barrier()

        w0 = sid * (NP // 16)
        for off, sz in _stripe_chunks(WCH):
            pltpu.sync_copy(acc.at[pl.ds(w0 + off, sz)],
                            zw_v.at[pl.ds(0, sz)])
            pltpu.sync_copy(zw_v.at[pl.ds(0, sz)],
                            out_hbm.at[pl.ds(cid * NP + w0 + off, sz)])

    return k(m, dsts, zeros8)


def _sc_count(dsts, cvals, zeros8):
    @functools.partial(
        pl.kernel,
        out_type=_sds((2 * NP, 8)),
        mesh=_mesh(),
        scratch_types=[
            pltpu.VMEM((GRP, CH), jnp.int32),
            pltpu.VMEM((CH, 8), F32),
            pltpu.VMEM((WCH, 8), F32),
            pltpu.VMEM_SHARED((ACC_R, 8), F32),
            pltpu.SemaphoreType.DMA,
            pltpu.SemaphoreType.DMA,
        ],
        compiler_params=_SC_PARAMS,
    )
    def k(ds_hbm, cv_hbm, zr_hbm, out_hbm, idx_v, vals_v, zw_v, acc,
          seml, sems):
        cid = lax.axis_index("c")
        sid = lax.axis_index("s")
        pltpu.sync_copy(zr_hbm, zw_v)
        z0 = sid * (NP // 16)
        for off, sz in _stripe_chunks(WCH):
            pltpu.sync_copy(zw_v.at[pl.ds(0, sz)],
                            acc.at[pl.ds(z0 + off, sz)])
        pltpu.sync_copy(cv_hbm, vals_v)
        plsc.subcore_barrier()

        row0 = cid * (ROWS // NC) + sid * (ROWS // NWK)

        def grp(g, carry):
            r = row0 + g * GRP
            ci = pltpu.async_copy(ds_hbm.at[pl.ds(r, GRP)], idx_v, seml)
            ci.wait()
            cps = [pltpu.async_copy(vals_v, acc.at[idx_v.at[j]], sems,
                                    add=True)
                   for j in range(GRP)]
            for c in cps:
                c.wait()
            return carry

        lax.fori_loop(0, (ROWS // NWK) // GRP, grp, 0)
        plsc.subcore_barrier()

        w0 = sid * (NP // 16)
        for off, sz in _stripe_chunks(WCH):
            pltpu.sync_copy(acc.at[pl.ds(w0 + off, sz)],
                            zw_v.at[pl.ds(0, sz)])
            pltpu.sync_copy(zw_v.at[pl.ds(0, sz)],
                            out_hbm.at[pl.ds(cid * NP + w0 + off, sz)])

    return k(dsts, cvals, zeros8)


def _prep_conv(p, fdim, fout):
    w0, w1, w2 = p["W"]
    b0, b1, b2 = p["b"]
    wa = w0[:fdim] - w0[fdim:]
    wb = w0[fdim:]
    eye4 = jnp.eye(4, dtype=F32)
    if fout < 32:
        w2 = jnp.concatenate([w2, jnp.zeros((32, 32 - fout), F32)],
                             axis=1)
        b2 = jnp.concatenate([b2, jnp.zeros((32 - fout,), F32)])
    d = {
        "bdw2": jnp.kron(eye4, w1),
        "b2t": jnp.tile(b1.reshape(1, 32), (1, 4)),
        "bdw3": jnp.kron(eye4, w2),
        "b3t": jnp.tile(b2.reshape(1, 32), (1, 4)),
    }
    if fdim == 4:
        d["ta"] = jnp.kron(eye4, wa)
        d["tb"] = jnp.kron(eye4, wb)
        d["tbias"] = jnp.tile(b0.reshape(1, 32), (1, 4))
    elif fdim == 32:
        eye8 = jnp.eye(8, dtype=F32)
        d["ta"] = jnp.concatenate(
            [jnp.kron(eye8, wa[:16]), jnp.kron(eye8, wa[16:])], axis=0)
        d["tb"] = jnp.concatenate(
            [jnp.kron(eye8, wb[:16]), jnp.kron(eye8, wb[16:])], axis=0)
        d["tbias"] = jnp.tile(b0.reshape(1, 32), (1, 8))
    else:
        eye16 = jnp.eye(16, dtype=F32)
        wap = jnp.concatenate([wa, jnp.zeros((6, 32), F32)], axis=0)
        wbp = jnp.concatenate([wb, jnp.zeros((6, 32), F32)], axis=0)
        d["ta"] = jnp.kron(eye16, wap)
        d["tb"] = jnp.kron(eye16, wbp)
        d["tbias"] = jnp.tile(b0.reshape(1, 32), (1, 16))
    return d


def kernel(x, edge_index, params):
    src = edge_index[0].astype(jnp.int32)
    dst = edge_index[1].astype(jnp.int32)
    npad = EPAD - NE
    pad_g = (jnp.arange(npad, dtype=jnp.int32) * 97) % NN
    pad_s = NN + (jnp.arange(npad, dtype=jnp.int32) % 8)
    dstg = jnp.concatenate([dst, pad_g]).reshape(ROWS, CH)
    srcg = jnp.concatenate([src, pad_g]).reshape(ROWS, CH)
    dsts = jnp.concatenate([dst, pad_s]).reshape(ROWS, CH)

    x128 = x.reshape(NN // 32, 128)
    x4 = x.reshape(NN // 4, 16)
    msel = (jnp.arange(128, dtype=jnp.int32)[:, None] % 4
            == jnp.arange(4, dtype=jnp.int32)[None, :]).astype(F32)
    bn2 = jnp.stack([params["bn"]["gamma"], params["bn"]["beta"]])
    zeros16 = jnp.zeros((512, 16), F32)
    zeros8 = jnp.zeros((WCH, 8), F32)
    cvals = (jnp.arange(8, dtype=jnp.int32)[None, :] == 0
             ).astype(F32) * jnp.ones((CH, 1), F32)

    lanes = jnp.arange(128)
    bc8 = jnp.kron(jnp.eye(16, dtype=F32),
                   jnp.zeros((8, 8), F32).at[0].set(1.0))
    d0 = jnp.zeros((128, 128), F32).at[(lanes // 16) * 8, lanes].set(1.0)
    d1 = jnp.zeros((128, 128), F32).at[64 + (lanes // 16) * 8,
                                       lanes].set(1.0)
    l64 = jnp.arange(64)
    sel = jnp.zeros((128, 64), F32).at[(l64 // 4) * 8 + l64 % 4,
                                       l64].set(1.0)

    e1 = _prep_conv(params["enc1"], 4, 32)
    e2 = _prep_conv(params["enc2"], 32, 2)
    dc1 = _prep_conv(params["dec1"], 2, 32)
    dc2 = _prep_conv(params["dec2"], 32, 4)

    cnt2 = _sc_count(dsts, cvals, zeros8)
    dep = cnt2[:8]
    c0 = cnt2[:NP].reshape(NP // 16, 128)
    c1 = cnt2[NP:].reshape(NP // 16, 128)
    inv16, inv8w = _cnt_expand(c0, c1, bc8, d0, d1)
    inv8 = inv8w.reshape(NP // 8, 128)

    stats = _stats(x128)
    p, q = _tables0(x4, stats, msel, bn2, e1["ta"], e1["tb"], e1["tbias"])

    def as_tab(t):
        return t.reshape(-1, 32)

    def g128(g):
        return g.reshape(EPAD // 4, 128)

    g1, g2 = _sc_gather(as_tab(p), as_tab(q), dstg, srcg, dep)
    m = _mlp(True, g128(g1), g128(g2), e1["bdw2"], e1["b2t"],
             e1["bdw3"], e1["b3t"])
    slo, shi = _sc_scatter_wide(m.reshape(EPAD, 32), dsts, zeros16)
    p, q = _tables_wide(slo.reshape(NP // 8, 128),
                        shi.reshape(NP // 8, 128), inv8,
                        e2["ta"], e2["tb"], e2["tbias"])
    g1, g2 = _sc_gather(as_tab(p), as_tab(q), dstg, srcg, dep)
    m = _mlp(True, g128(g1), g128(g2), e2["bdw2"], e2["b2t"],
             e2["bdw3"], e2["b3t"])
    s8 = _sc_scatter_narrow(m.reshape(EPAD, 32), dsts, zeros8)
    p, q = _tables_narrow(s8[:NP].reshape(NP // 16, 128),
                          s8[NP:].reshape(NP // 16, 128), inv16,
                          dc1["ta"], dc1["tb"], dc1["tbias"])
    g1, g2 = _sc_gather(as_tab(p), as_tab(q), dstg, srcg, dep)
    m = _mlp(True, g128(g1), g128(g2), dc1["bdw2"], dc1["b2t"],
             dc1["bdw3"], dc1["b3t"])
    slo, shi = _sc_scatter_wide(m.reshape(EPAD, 32), dsts, zeros16)
    p, q = _tables_wide(slo.reshape(NP // 8, 128),
                        shi.reshape(NP // 8, 128), inv8,
                        dc2["ta"], dc2["tb"], dc2["tbias"])
    g1, g2 = _sc_gather(as_tab(p), as_tab(q), dstg, srcg, dep)
    m = _mlp(False, g128(g1), g128(g2), dc2["bdw2"], dc2["b2t"],
             dc2["bdw3"], dc2["b3t"])
    s8 = _sc_scatter_narrow(m.reshape(EPAD, 32), dsts, zeros8)

    out = _final(s8[:NP].reshape(NP // 16, 128),
                 s8[NP:].reshape(NP // 16, 128), inv16, sel)
    return out.reshape(NP, 4)[:NN]


# speedup vs baseline: 10.9873x; 1.2455x over previous
import functools

import jax
import jax.numpy as jnp
from jax import lax
from jax.experimental import pallas as pl
from jax.experimental.pallas import tpu as pltpu
from jax.experimental.pallas import tpu_sc as plsc

NN = 100000
NE = 1600000
NC, NS = 2, 16
NWK = NC * NS
CH = 128
GRP = 8
GE = CH * GRP
EPAD = NWK * 392 * CH
ROWS = EPAD // CH
NP = 100096
ACC_R = NP
WCH = 2048
F32 = jnp.float32

_SC_PARAMS = pltpu.CompilerParams(use_tc_tiling_on_sc=False)


def _sds(shape):
    return jax.ShapeDtypeStruct(shape, F32)


def _mesh():
    return plsc.VectorSubcoreMesh(core_axis_name="c", subcore_axis_name="s")


def _stripe_chunks(n):
    stripe = NP // 16
    out = [(i * n, n) for i in range(stripe // n)]
    if stripe % n:
        out.append((stripe // n * n, stripe % n))
    return out


def _stats(x128):
    def body(x_ref, o_ref):
        x = x_ref[...]
        s = jnp.sum(x, axis=0, keepdims=True)
        s2 = jnp.sum(x * x, axis=0, keepdims=True)
        o_ref[...] = jnp.concatenate(
            [s, s2, jnp.zeros((6, 128), F32)], axis=0)
    return pl.pallas_call(body, out_shape=_sds((8, 128)))(x128)


def _state0(x4, stats, msel, bn2, exp16):
    def body(x_ref, st_ref, ms_ref, bn_ref, e_ref, h_ref):
        st = st_ref[...]
        ms = ms_ref[...]
        mean = jnp.dot(st[0:1], ms, preferred_element_type=F32,
                       precision=lax.Precision.HIGHEST) / NN
        ex2 = jnp.dot(st[1:2], ms, preferred_element_type=F32,
                      precision=lax.Precision.HIGHEST) / NN
        var = ex2 - mean * mean
        m4 = jnp.tile(mean, (1, 4))
        v4 = jnp.tile(var, (1, 4))
        g4 = jnp.tile(bn_ref[0:1], (1, 4))
        b4 = jnp.tile(bn_ref[1:2], (1, 4))
        h = (x_ref[...] - m4) / jnp.sqrt(v4 + 1e-5) * g4 + b4
        h_ref[...] = jnp.dot(h, e_ref[...], preferred_element_type=F32,
                             precision=lax.Precision.HIGHEST)
    return pl.pallas_call(
        body,
        grid=(5,),
        in_specs=[
            pl.BlockSpec((NN // 20, 16), lambda i: (i, 0)),
            pl.BlockSpec((8, 128), lambda i: (0, 0)),
            pl.BlockSpec((128, 4), lambda i: (0, 0)),
            pl.BlockSpec((2, 4), lambda i: (0, 0)),
            pl.BlockSpec((16, 128), lambda i: (0, 0)),
        ],
        out_specs=pl.BlockSpec((NN // 20, 128), lambda i: (i, 0)),
        out_shape=_sds((NN // 4, 128)),
    )(x4, stats, msel, bn2, exp16)


def _cnt_expand(c0, c1, bc8, d0, d1):
    def body(c0_ref, c1_ref, bc_ref, d0_ref, d1_ref, i16_ref, i8_ref):
        cnt = jnp.maximum(c0_ref[...] + c1_ref[...], 1.0)
        i16 = jnp.dot(cnt, bc_ref[...], preferred_element_type=F32,
                      precision=lax.Precision.HIGHEST)
        i16_ref[...] = i16
        a8 = jnp.dot(i16, d0_ref[...], preferred_element_type=F32,
                     precision=lax.Precision.HIGHEST)
        b8 = jnp.dot(i16, d1_ref[...], preferred_element_type=F32,
                     precision=lax.Precision.HIGHEST)
        i8_ref[...] = jnp.concatenate([a8, b8], axis=1)
    return pl.pallas_call(
        body,
        grid=(2,),
        in_specs=[
            pl.BlockSpec((3128, 128), lambda i: (i, 0)),
            pl.BlockSpec((3128, 128), lambda i: (i, 0)),
            pl.BlockSpec((128, 128), lambda i: (0, 0)),
            pl.BlockSpec((128, 128), lambda i: (0, 0)),
            pl.BlockSpec((128, 128), lambda i: (0, 0)),
        ],
        out_specs=[pl.BlockSpec((3128, 128), lambda i: (i, 0)),
                   pl.BlockSpec((3128, 256), lambda i: (i, 0))],
        out_shape=[_sds((NP // 16, 128)), _sds((NP // 16, 256))],
    )(c0, c1, bc8, d0, d1)


def _state_wide(slo8, shi8, c8, perm):
    def body(lo_ref, hi_ref, c_ref, p_ref, h_ref):
        c = c_ref[...]
        h = jnp.concatenate([lo_ref[...] / c, hi_ref[...] / c], axis=1)
        h_ref[...] = jnp.dot(h, p_ref[...], preferred_element_type=F32,
                             precision=lax.Precision.HIGHEST)
    return pl.pallas_call(
        body,
        grid=(4,),
        in_specs=[
            pl.BlockSpec((3128, 128), lambda i: (i, 0)),
            pl.BlockSpec((3128, 128), lambda i: (i, 0)),
            pl.BlockSpec((3128, 128), lambda i: (i, 0)),
            pl.BlockSpec((256, 256), lambda i: (0, 0)),
        ],
        out_specs=pl.BlockSpec((3128, 256), lambda i: (i, 0)),
        out_shape=_sds((NP // 8, 256)),
    )(slo8, shi8, c8, perm)


def _state_narrow(s0, s1, c16, expn):
    def body(s0_ref, s1_ref, c_ref, e_ref, h_ref):
        h = (s0_ref[...] + s1_ref[...]) / c_ref[...]
        h_ref[...] = jnp.dot(h, e_ref[...], preferred_element_type=F32,
                             precision=lax.Precision.HIGHEST)
    return pl.pallas_call(
        body,
        grid=(2,),
        in_specs=[
            pl.BlockSpec((3128, 128), lambda i: (i, 0)),
            pl.BlockSpec((3128, 128), lambda i: (i, 0)),
            pl.BlockSpec((3128, 128), lambda i: (i, 0)),
            pl.BlockSpec((128, 512), lambda i: (0, 0)),
        ],
        out_specs=pl.BlockSpec((3128, 512), lambda i: (i, 0)),
        out_shape=_sds((NP // 16, 512)),
    )(s0, s1, c16, expn)


def _mlp(last_relu, g1, g2, w1k, b1t, bdw2, b2t, bdw3, b3t):
    def body(g1_ref, g2_ref, w1_ref, b1_ref, w2_ref, b2_ref, w3_ref,
             b3_ref, m_ref):
        xi = g1_ref[...]
        feat = jnp.concatenate([xi, g2_ref[...] - xi], axis=1)
        h1 = jnp.maximum(
            jnp.dot(feat, w1_ref[...], preferred_element_type=F32)
            + b1_ref[...], 0.0)
        h2 = jnp.maximum(
            jnp.dot(h1, w2_ref[...], preferred_element_type=F32)
            + b2_ref[...], 0.0)
        m = jnp.dot(h2, w3_ref[...], preferred_element_type=F32) \
            + b3_ref[...]
        if last_relu:
            m = jnp.maximum(m, 0.0)
        m_ref[...] = m
    return pl.pallas_call(
        body,
        grid=(392,),
        in_specs=[
            pl.BlockSpec((1024, 128), lambda i: (i, 0)),
            pl.BlockSpec((1024, 128), lambda i: (i, 0)),
            pl.BlockSpec((256, 128), lambda i: (0, 0)),
            pl.BlockSpec((1, 128), lambda i: (0, 0)),
            pl.BlockSpec((128, 128), lambda i: (0, 0)),
            pl.BlockSpec((1, 128), lambda i: (0, 0)),
            pl.BlockSpec((128, 128), lambda i: (0, 0)),
            pl.BlockSpec((1, 128), lambda i: (0, 0)),
        ],
        out_specs=pl.BlockSpec((1024, 128), lambda i: (i, 0)),
        out_shape=_sds((EPAD // 4, 128)),
    )(g1, g2, w1k, b1t, bdw2, b2t, bdw3, b3t)


def _final(s0, s1, c16, sel):
    def body(s0_ref, s1_ref, c_ref, sel_ref, o_ref):
        v = (s0_ref[...] + s1_ref[...]) / c_ref[...]
        o_ref[...] = jnp.dot(v, sel_ref[...], preferred_element_type=F32,
                             precision=lax.Precision.HIGHEST)
    return pl.pallas_call(
        body,
        grid=(2,),
        in_specs=[
            pl.BlockSpec((3128, 128), lambda i: (i, 0)),
            pl.BlockSpec((3128, 128), lambda i: (i, 0)),
            pl.BlockSpec((3128, 128), lambda i: (i, 0)),
            pl.BlockSpec((128, 64), lambda i: (0, 0)),
        ],
        out_specs=pl.BlockSpec((3128, 64), lambda i: (i, 0)),
        out_shape=_sds((NP // 16, 64)),
    )(s0, s1, c16, sel)


def _sc_gather(ptab, qtab, dstg, srcg, dep):
    @functools.partial(
        pl.kernel,
        out_type=[_sds((EPAD, 32))] * 2,
        mesh=_mesh(),
        scratch_types=[
            pltpu.VMEM((GRP, CH), jnp.int32),
            pltpu.VMEM((GRP, CH), jnp.int32),
            pltpu.VMEM((GE, 32), F32),
            pltpu.VMEM((GE, 32), F32),
            pltpu.SemaphoreType.DMA,
            pltpu.SemaphoreType.DMA,
            pltpu.SemaphoreType.DMA,
        ],
        compiler_params=_SC_PARAMS,
    )
    def k(p_hbm, q_hbm, dg_hbm, sg_hbm, dep_hbm, g1_hbm, g2_hbm,
          idxd_v, idxs_v, rowsp_v, rowsq_v, seml, semg, semw):
        wid = lax.axis_index("s") * NC + lax.axis_index("c")
        row0 = wid * (ROWS // NWK)

        def grp(g, carry):
            r = row0 + g * GRP
            base = r * CH
            ci = pltpu.async_copy(dg_hbm.at[pl.ds(r, GRP)], idxd_v, seml)
            cs = pltpu.async_copy(sg_hbm.at[pl.ds(r, GRP)], idxs_v, seml)
            ci.wait()
            cps_p = [pltpu.async_copy(p_hbm.at[idxd_v.at[j]],
                                      rowsp_v.at[pl.ds(j * CH, CH)],
                                      semg)
                     for j in range(GRP)]
            cs.wait()
            cps_q = [pltpu.async_copy(q_hbm.at[idxs_v.at[j]],
                                      rowsq_v.at[pl.ds(j * CH, CH)],
                                      semg)
                     for j in range(GRP)]
            for c in cps_p:
                c.wait()
            w1 = pltpu.async_copy(rowsp_v, g1_hbm.at[pl.ds(base, GE)],
                                  semw)
            for c in cps_q:
                c.wait()
            w2 = pltpu.async_copy(rowsq_v, g2_hbm.at[pl.ds(base, GE)],
                                  semw)
            w1.wait()
            w2.wait()
            return carry

        lax.fori_loop(0, (ROWS // NWK) // GRP, grp, 0)

    return k(ptab, qtab, dstg, srcg, dep)


def _sc_scatter_wide(m, dsts, zeros16):
    @functools.partial(
        pl.kernel,
        out_type=[_sds((NP, 16))] * 2,
        mesh=_mesh(),
        scratch_types=[
            pltpu.VMEM((GRP, CH), jnp.int32),
            pltpu.VMEM((GE, 16), F32),
            pltpu.VMEM((512, 16), F32),
            pltpu.VMEM_SHARED((ACC_R, 16), F32),
            pltpu.SemaphoreType.DMA,
            pltpu.SemaphoreType.DMA,
        ],
        compiler_params=_SC_PARAMS,
    )
    def k(m_hbm, ds_hbm, zr_hbm, slo_hbm, shi_hbm,
          idx_v, vals_v, zw_v, acc, seml, sems):
        cid = lax.axis_index("c")
        sid = lax.axis_index("s")
        pltpu.sync_copy(zr_hbm, zw_v)
        z0 = sid * (NP // 16)
        for off, sz in _stripe_chunks(512):
            pltpu.sync_copy(zw_v.at[pl.ds(0, sz)],
                            acc.at[pl.ds(z0 + off, sz)])
        plsc.subcore_barrier()

        row0 = sid * (ROWS // NS)

        def grp(g, carry):
            r = row0 + g * GRP
            ci = pltpu.async_copy(ds_hbm.at[pl.ds(r, GRP)], idx_v, seml)

            @pl.when(cid == 0)
            def _():
                pltpu.async_copy(
                    m_hbm.at[pl.ds(r * CH, GE), pl.ds(0, 16)], vals_v,
                    seml)

            @pl.when(cid == 1)
            def _():
                pltpu.async_copy(
                    m_hbm.at[pl.ds(r * CH, GE), pl.ds(16, 16)], vals_v,
                    seml)

            ci.wait()
            cv = pltpu.make_async_copy(
                m_hbm.at[pl.ds(r * CH, GE), pl.ds(0, 16)], vals_v, seml)
            cv.wait()
            cps = [pltpu.async_copy(vals_v.at[pl.ds(j * CH, CH)],
                                    acc.at[idx_v.at[j]], sems, add=True)
                   for j in range(GRP)]
            for c in cps:
                c.wait()
            return carry

        lax.fori_loop(0, (ROWS // NS) // GRP, grp, 0)
        plsc.subcore_barrier()

        w0 = sid * (NP // 16)
        for off, sz in _stripe_chunks(512):
            pltpu.sync_copy(acc.at[pl.ds(w0 + off, sz)],
                            zw_v.at[pl.ds(0, sz)])

            @pl.when(cid == 0)
            def _():
                pltpu.sync_copy(zw_v.at[pl.ds(0, sz)],
                                slo_hbm.at[pl.ds(w0 + off, sz)])

            @pl.when(cid == 1)
            def _():
                pltpu.sync_copy(zw_v.at[pl.ds(0, sz)],
                                shi_hbm.at[pl.ds(w0 + off, sz)])

    return k(m, dsts, zeros16)


def _sc_scatter_narrow(m, dsts, zeros8):
    @functools.partial(
        pl.kernel,
        out_type=_sds((2 * NP, 8)),
        mesh=_mesh(),
        scratch_types=[
            pltpu.VMEM((GRP, CH), jnp.int32),
            pltpu.VMEM((GE, 8), F32),
            pltpu.VMEM((WCH, 8), F32),
            pltpu.VMEM_SHARED((ACC_R, 8), F32),
            pltpu.SemaphoreType.DMA,
            pltpu.SemaphoreType.DMA,
        ],
        compiler_params=_SC_PARAMS,
    )
    def k(m_hbm, ds_hbm, zr_hbm, out_hbm, idx_v, vals_v, zw_v, acc,
          seml, sems):
        cid = lax.axis_index("c")
        sid = lax.axis_index("s")
        pltpu.sync_copy(zr_hbm, zw_v)
        z0 = sid * (NP // 16)
        for off, sz in _stripe_chunks(WCH):
            pltpu.sync_copy(zw_v.at[pl.ds(0, sz)],
                            acc.at[pl.ds(z0 + off, sz)])
        plsc.subcore_barrier()

        row0 = cid * (ROWS // NC) + sid * (ROWS // NWK)

        def grp(g, carry):
            r = row0 + g * GRP
            ci = pltpu.async_copy(ds_hbm.at[pl.ds(r, GRP)], idx_v, seml)
            cv = pltpu.async_copy(
                m_hbm.at[pl.ds(r * CH, GE), pl.ds(0, 8)], vals_v, seml)
            ci.wait()
            cv.wait()
            cps = [pltpu.async_copy(vals_v.at[pl.ds(j * CH, CH)],
                                    acc.at[idx_v.at[j]], sems, add=True)
                   for j in range(GRP)]
            for c in cps:
                c.wait()
            return carry

        lax.fori_loop(0, (ROWS // NWK) // GRP, grp, 0)
        plsc.subcore_barrier()

        w0 = sid * (NP // 16)
        for off, sz in _stripe_chunks(WCH):
            pltpu.sync_copy(acc.at[pl.ds(w0 + off, sz)],
                            zw_v.at[pl.ds(0, sz)])
            pltpu.sync_copy(zw_v.at[pl.ds(0, sz)],
                            out_hbm.at[pl.ds(cid * NP + w0 + off, sz)])

    return k(m, dsts, zeros8)


def _sc_count(dsts, cvals, zeros8):
    @functools.partial(
        pl.kernel,
        out_type=_sds((2 * NP, 8)),
        mesh=_mesh(),
        scratch_types=[
            pltpu.VMEM((GRP, CH), jnp.int32),
            pltpu.VMEM((CH, 8), F32),
            pltpu.VMEM((WCH, 8), F32),
            pltpu.VMEM_SHARED((ACC_R, 8), F32),
            pltpu.SemaphoreType.DMA,
            pltpu.SemaphoreType.DMA,
        ],
        compiler_params=_SC_PARAMS,
    )
    def k(ds_hbm, cv_hbm, zr_hbm, out_hbm, idx_v, vals_v, zw_v, acc,
          seml, sems):
        cid = lax.axis_index("c")
        sid = lax.axis_index("s")
        pltpu.sync_copy(zr_hbm, zw_v)
        z0 = sid * (NP // 16)
        for off, sz in _stripe_chunks(WCH):
            pltpu.sync_copy(zw_v.at[pl.ds(0, sz)],
                            acc.at[pl.ds(z0 + off, sz)])
        pltpu.sync_copy(cv_hbm, vals_v)
        plsc.subcore_barrier()

        row0 = cid * (ROWS // NC) + sid * (ROWS // NWK)

        def grp(g, carry):
            r = row0 + g * GRP
            ci = pltpu.async_copy(ds_hbm.at[pl.ds(r, GRP)], idx_v, seml)
            ci.wait()
            cps = [pltpu.async_copy(vals_v, acc.at[idx_v.at[j]], sems,
                                    add=True)
                   for j in range(GRP)]
            for c in cps:
                c.wait()
            return carry

        lax.fori_loop(0, (ROWS // NWK) // GRP, grp, 0)
        plsc.subcore_barrier()

        w0 = sid * (NP // 16)
        for off, sz in _stripe_chunks(WCH):
            pltpu.sync_copy(acc.at[pl.ds(w0 + off, sz)],
                            zw_v.at[pl.ds(0, sz)])
            pltpu.sync_copy(zw_v.at[pl.ds(0, sz)],
                            out_hbm.at[pl.ds(cid * NP + w0 + off, sz)])

    return k(dsts, cvals, zeros8)


def _prep_conv(p, fdim, fout):
    w0, w1, w2 = p["W"]
    b0, b1, b2 = p["b"]
    eye4 = jnp.eye(4, dtype=F32)
    w1a = jnp.zeros((32, 32), F32).at[:fdim].set(w0[:fdim])
    w1b = jnp.zeros((32, 32), F32).at[:fdim].set(w0[fdim:])
    if fout < 32:
        w2 = jnp.concatenate([w2, jnp.zeros((32, 32 - fout), F32)],
                             axis=1)
        b2 = jnp.concatenate([b2, jnp.zeros((32 - fout,), F32)])
    return {
        "w1k": jnp.concatenate(
            [jnp.kron(eye4, w1a), jnp.kron(eye4, w1b)], axis=0),
        "b1t": jnp.tile(b0.reshape(1, 32), (1, 4)),
        "bdw2": jnp.kron(eye4, w1),
        "b2t": jnp.tile(b1.reshape(1, 32), (1, 4)),
        "bdw3": jnp.kron(eye4, w2),
        "b3t": jnp.tile(b2.reshape(1, 32), (1, 4)),
    }


def kernel(x, edge_index, params):
    src = edge_index[0].astype(jnp.int32)
    dst = edge_index[1].astype(jnp.int32)
    npad = EPAD - NE
    pad_g = (jnp.arange(npad, dtype=jnp.int32) * 97) % NN
    pad_s = NN + (jnp.arange(npad, dtype=jnp.int32) % 8)
    dstg = jnp.concatenate([dst, pad_g]).reshape(ROWS, CH)
    srcg = jnp.concatenate([src, pad_g]).reshape(ROWS, CH)
    dsts = jnp.concatenate([dst, pad_s]).reshape(ROWS, CH)

    x128 = x.reshape(NN // 32, 128)
    x4 = x.reshape(NN // 4, 16)
    msel = (jnp.arange(128, dtype=jnp.int32)[:, None] % 4
            == jnp.arange(4, dtype=jnp.int32)[None, :]).astype(F32)
    bn2 = jnp.stack([params["bn"]["gamma"], params["bn"]["beta"]])
    zeros16 = jnp.zeros((512, 16), F32)
    zeros8 = jnp.zeros((WCH, 8), F32)
    cvals = (jnp.arange(8, dtype=jnp.int32)[None, :] == 0
             ).astype(F32) * jnp.ones((CH, 1), F32)

    lanes = jnp.arange(128)
    bc8 = jnp.kron(jnp.eye(16, dtype=F32),
                   jnp.zeros((8, 8), F32).at[0].set(1.0))
    d0 = jnp.zeros((128, 128), F32).at[(lanes // 16) * 8, lanes].set(1.0)
    d1 = jnp.zeros((128, 128), F32).at[64 + (lanes // 16) * 8,
                                       lanes].set(1.0)
    l64 = jnp.arange(64)
    sel = jnp.zeros((128, 64), F32).at[(l64 // 4) * 8 + l64 % 4,
                                       l64].set(1.0)
    exp16 = jnp.kron(jnp.eye(4, dtype=F32),
                     jnp.concatenate(
                         [jnp.eye(4, dtype=F32),
                          jnp.zeros((4, 28), F32)], axis=1))
    t8 = lanes // 16
    i16 = lanes % 16
    perm = jnp.zeros((256, 256), F32)
    perm = perm.at[lanes, t8 * 32 + i16].set(1.0)
    perm = perm.at[128 + lanes, t8 * 32 + 16 + i16].set(1.0)
    t16 = lanes // 8
    i8 = lanes % 8
    expn = jnp.zeros((128, 512), F32)
    expn = expn.at[lanes, t16 * 32 + i8].set(
        (i8 < 2).astype(F32))

    e1 = _prep_conv(params["enc1"], 4, 32)
    e2 = _prep_conv(params["enc2"], 32, 2)
    dc1 = _prep_conv(params["dec1"], 2, 32)
    dc2 = _prep_conv(params["dec2"], 32, 4)

    cnt2 = _sc_count(dsts, cvals, zeros8)
    dep = cnt2[:8]
    cp0 = cnt2[:NP].reshape(NP // 16, 128)
    cp1 = cnt2[NP:].reshape(NP // 16, 128)
    c16, c8w = _cnt_expand(cp0, cp1, bc8, d0, d1)
    c8 = c8w.reshape(NP // 8, 128)

    stats = _stats(x128)
    h = _state0(x4, stats, msel, bn2, exp16)

    def as_tab(t):
        return t.reshape(-1, 32)

    def g128(g):
        return g.reshape(EPAD // 4, 128)

    def conv(hpk, prep, last_relu):
        tab = as_tab(hpk)
        g1, g2 = _sc_gather(tab, tab, dstg, srcg, dep)
        return _mlp(last_relu, g128(g1), g128(g2), prep["w1k"],
                    prep["b1t"], prep["bdw2"], prep["b2t"],
                    prep["bdw3"], prep["b3t"])

    m = conv(h, e1, True)
    slo, shi = _sc_scatter_wide(m.reshape(EPAD, 32), dsts, zeros16)
    h = _state_wide(slo.reshape(NP // 8, 128),
                    shi.reshape(NP // 8, 128), c8, perm)
    m = conv(h, e2, True)
    s8 = _sc_scatter_narrow(m.reshape(EPAD, 32), dsts, zeros8)
    h = _state_narrow(s8[:NP].reshape(NP // 16, 128),
                      s8[NP:].reshape(NP // 16, 128), c16, expn)
    m = conv(h, dc1, True)
    slo, shi = _sc_scatter_wide(m.reshape(EPAD, 32), dsts, zeros16)
    h = _state_wide(slo.reshape(NP // 8, 128),
                    shi.reshape(NP // 8, 128), c8, perm)
    m = conv(h, dc2, False)
    s8 = _sc_scatter_narrow(m.reshape(EPAD, 32), dsts, zeros8)

    out = _final(s8[:NP].reshape(NP // 16, 128),
                 s8[NP:].reshape(NP // 16, 128), c16, sel)
    return out.reshape(NP, 4)[:NN]


# speedup vs baseline: 11.3126x; 1.0296x over previous
import functools

import jax
import jax.numpy as jnp
from jax import lax
from jax.experimental import pallas as pl
from jax.experimental.pallas import tpu as pltpu
from jax.experimental.pallas import tpu_sc as plsc

NN = 100000
NE = 1600000
NC, NS = 2, 16
NWK = NC * NS
CH = 128
GRP = 8
GE = CH * GRP
EPAD = NWK * 392 * CH
ROWS = EPAD // CH
NP = 100096
ACC_R = NP
WCH = 2048
F32 = jnp.float32

_SC_PARAMS = pltpu.CompilerParams(use_tc_tiling_on_sc=False)


def _sds(shape):
    return jax.ShapeDtypeStruct(shape, F32)


def _mesh():
    return plsc.VectorSubcoreMesh(core_axis_name="c", subcore_axis_name="s")


def _stripe_chunks(n):
    stripe = NP // 16
    out = [(i * n, n) for i in range(stripe // n)]
    if stripe % n:
        out.append((stripe // n * n, stripe % n))
    return out


def _stats(x128):
    def body(x_ref, o_ref):
        x = x_ref[...]
        s = jnp.sum(x, axis=0, keepdims=True)
        s2 = jnp.sum(x * x, axis=0, keepdims=True)
        o_ref[...] = jnp.concatenate(
            [s, s2, jnp.zeros((6, 128), F32)], axis=0)
    return pl.pallas_call(body, out_shape=_sds((8, 128)))(x128)


def _state0(x4, stats, msel, bn2, exp16):
    def body(x_ref, st_ref, ms_ref, bn_ref, e_ref, h_ref):
        st = st_ref[...]
        ms = ms_ref[...]
        mean = jnp.dot(st[0:1], ms, preferred_element_type=F32,
                       precision=lax.Precision.HIGHEST) / NN
        ex2 = jnp.dot(st[1:2], ms, preferred_element_type=F32,
                      precision=lax.Precision.HIGHEST) / NN
        var = ex2 - mean * mean
        m4 = jnp.tile(mean, (1, 4))
        v4 = jnp.tile(var, (1, 4))
        g4 = jnp.tile(bn_ref[0:1], (1, 4))
        b4 = jnp.tile(bn_ref[1:2], (1, 4))
        h = (x_ref[...] - m4) / jnp.sqrt(v4 + 1e-5) * g4 + b4
        h_ref[...] = jnp.dot(h, e_ref[...], preferred_element_type=F32,
                             precision=lax.Precision.HIGHEST)
    return pl.pallas_call(
        body,
        grid=(5,),
        in_specs=[
            pl.BlockSpec((NN // 20, 16), lambda i: (i, 0)),
            pl.BlockSpec((8, 128), lambda i: (0, 0)),
            pl.BlockSpec((128, 4), lambda i: (0, 0)),
            pl.BlockSpec((2, 4), lambda i: (0, 0)),
            pl.BlockSpec((16, 128), lambda i: (0, 0)),
        ],
        out_specs=pl.BlockSpec((NN // 20, 128), lambda i: (i, 0)),
        out_shape=_sds((NN // 4, 128)),
    )(x4, stats, msel, bn2, exp16)


def _cnt_expand(c0, c1, bc8, d0, d1):
    def body(c0_ref, c1_ref, bc_ref, d0_ref, d1_ref, i16_ref, i8_ref):
        cnt = jnp.maximum(c0_ref[...] + c1_ref[...], 1.0)
        i16 = jnp.dot(cnt, bc_ref[...], preferred_element_type=F32,
                      precision=lax.Precision.HIGHEST)
        i16_ref[...] = i16
        a8 = jnp.dot(i16, d0_ref[...], preferred_element_type=F32,
                     precision=lax.Precision.HIGHEST)
        b8 = jnp.dot(i16, d1_ref[...], preferred_element_type=F32,
                     precision=lax.Precision.HIGHEST)
        i8_ref[...] = jnp.concatenate([a8, b8], axis=1)
    return pl.pallas_call(
        body,
        grid=(2,),
        in_specs=[
            pl.BlockSpec((3128, 128), lambda i: (i, 0)),
            pl.BlockSpec((3128, 128), lambda i: (i, 0)),
            pl.BlockSpec((128, 128), lambda i: (0, 0)),
            pl.BlockSpec((128, 128), lambda i: (0, 0)),
            pl.BlockSpec((128, 128), lambda i: (0, 0)),
        ],
        out_specs=[pl.BlockSpec((3128, 128), lambda i: (i, 0)),
                   pl.BlockSpec((3128, 256), lambda i: (i, 0))],
        out_shape=[_sds((NP // 16, 128)), _sds((NP // 16, 256))],
    )(c0, c1, bc8, d0, d1)


def _state_wide(slo8, shi8, c8, perm):
    def body(lo_ref, hi_ref, c_ref, p_ref, h_ref):
        c = c_ref[...]
        h = jnp.concatenate([lo_ref[...] / c, hi_ref[...] / c], axis=1)
        h_ref[...] = jnp.dot(h, p_ref[...], preferred_element_type=F32,
                             precision=lax.Precision.HIGHEST)
    return pl.pallas_call(
        body,
        grid=(4,),
        in_specs=[
            pl.BlockSpec((3128, 128), lambda i: (i, 0)),
            pl.BlockSpec((3128, 128), lambda i: (i, 0)),
            pl.BlockSpec((3128, 128), lambda i: (i, 0)),
            pl.BlockSpec((256, 256), lambda i: (0, 0)),
        ],
        out_specs=pl.BlockSpec((3128, 256), lambda i: (i, 0)),
        out_shape=_sds((NP // 8, 256)),
    )(slo8, shi8, c8, perm)


def _state_narrow(s0, s1, c16, expn):
    def body(s0_ref, s1_ref, c_ref, e_ref, h_ref):
        h = (s0_ref[...] + s1_ref[...]) / c_ref[...]
        h_ref[...] = jnp.dot(h, e_ref[...], preferred_element_type=F32,
                             precision=lax.Precision.HIGHEST)
    return pl.pallas_call(
        body,
        grid=(2,),
        in_specs=[
            pl.BlockSpec((3128, 128), lambda i: (i, 0)),
            pl.BlockSpec((3128, 128), lambda i: (i, 0)),
            pl.BlockSpec((3128, 128), lambda i: (i, 0)),
            pl.BlockSpec((128, 512), lambda i: (0, 0)),
        ],
        out_specs=pl.BlockSpec((3128, 512), lambda i: (i, 0)),
        out_shape=_sds((NP // 16, 512)),
    )(s0, s1, c16, expn)


def _mlp(last_relu, g1, g2, w1k, b1t, bdw2, b2t, bdw3, b3t):
    def body(g1_ref, g2_ref, w1_ref, b1_ref, w2_ref, b2_ref, w3_ref,
             b3_ref, m_ref):
        xi = g1_ref[...]
        feat = jnp.concatenate([xi, g2_ref[...] - xi], axis=1)
        h1 = jnp.maximum(
            jnp.dot(feat, w1_ref[...], preferred_element_type=F32)
            + b1_ref[...], 0.0)
        h2 = jnp.maximum(
            jnp.dot(h1, w2_ref[...], preferred_element_type=F32)
            + b2_ref[...], 0.0)
        m = jnp.dot(h2, w3_ref[...], preferred_element_type=F32) \
            + b3_ref[...]
        if last_relu:
            m = jnp.maximum(m, 0.0)
        m_ref[...] = m
    return pl.pallas_call(
        body,
        grid=(392,),
        in_specs=[
            pl.BlockSpec((1024, 128), lambda i: (i, 0)),
            pl.BlockSpec((1024, 128), lambda i: (i, 0)),
            pl.BlockSpec((256, 128), lambda i: (0, 0)),
            pl.BlockSpec((1, 128), lambda i: (0, 0)),
            pl.BlockSpec((128, 128), lambda i: (0, 0)),
            pl.BlockSpec((1, 128), lambda i: (0, 0)),
            pl.BlockSpec((128, 128), lambda i: (0, 0)),
            pl.BlockSpec((1, 128), lambda i: (0, 0)),
        ],
        out_specs=pl.BlockSpec((1024, 128), lambda i: (i, 0)),
        out_shape=_sds((EPAD // 4, 128)),
    )(g1, g2, w1k, b1t, bdw2, b2t, bdw3, b3t)


def _final(s0, s1, c16, sel):
    def body(s0_ref, s1_ref, c_ref, sel_ref, o_ref):
        v = (s0_ref[...] + s1_ref[...]) / c_ref[...]
        o_ref[...] = jnp.dot(v, sel_ref[...], preferred_element_type=F32,
                             precision=lax.Precision.HIGHEST)
    return pl.pallas_call(
        body,
        grid=(2,),
        in_specs=[
            pl.BlockSpec((3128, 128), lambda i: (i, 0)),
            pl.BlockSpec((3128, 128), lambda i: (i, 0)),
            pl.BlockSpec((3128, 128), lambda i: (i, 0)),
            pl.BlockSpec((128, 64), lambda i: (0, 0)),
        ],
        out_specs=pl.BlockSpec((3128, 64), lambda i: (i, 0)),
        out_shape=_sds((NP // 16, 64)),
    )(s0, s1, c16, sel)


def _sc_gather(ptab, qtab, dstg, srcg, dep):
    @functools.partial(
        pl.kernel,
        out_type=[_sds((EPAD, 32))] * 2,
        mesh=_mesh(),
        scratch_types=[
            pltpu.VMEM((GRP, CH), jnp.int32),
            pltpu.VMEM((GRP, CH), jnp.int32),
            pltpu.VMEM((GE, 32), F32),
            pltpu.VMEM((GE, 32), F32),
            pltpu.SemaphoreType.DMA,
            pltpu.SemaphoreType.DMA,
            pltpu.SemaphoreType.DMA,
        ],
        compiler_params=_SC_PARAMS,
    )
    def k(p_hbm, q_hbm, dg_hbm, sg_hbm, dep_hbm, g1_hbm, g2_hbm,
          idxd_v, idxs_v, rowsp_v, rowsq_v, seml, semg, semw):
        wid = lax.axis_index("s") * NC + lax.axis_index("c")
        row0 = wid * (ROWS // NWK)

        def grp(g, carry):
            r = row0 + g * GRP
            base = r * CH
            ci = pltpu.async_copy(dg_hbm.at[pl.ds(r, GRP)], idxd_v, seml)
            cs = pltpu.async_copy(sg_hbm.at[pl.ds(r, GRP)], idxs_v, seml)
            ci.wait()
            cps_p = [pltpu.async_copy(p_hbm.at[idxd_v.at[j]],
                                      rowsp_v.at[pl.ds(j * CH, CH)],
                                      semg)
                     for j in range(GRP)]
            cs.wait()
            cps_q = [pltpu.async_copy(q_hbm.at[idxs_v.at[j]],
                                      rowsq_v.at[pl.ds(j * CH, CH)],
                                      semg)
                     for j in range(GRP)]
            for c in cps_p:
                c.wait()
            w1 = pltpu.async_copy(rowsp_v, g1_hbm.at[pl.ds(base, GE)],
                                  semw)
            for c in cps_q:
                c.wait()
            w2 = pltpu.async_copy(rowsq_v, g2_hbm.at[pl.ds(base, GE)],
                                  semw)
            w1.wait()
            w2.wait()
            return carry

        lax.fori_loop(0, (ROWS // NWK) // GRP, grp, 0)

    return k(ptab, qtab, dstg, srcg, dep)


def _sc_scatter_wide(m, dsts, zeros16):
    @functools.partial(
        pl.kernel,
        out_type=[_sds((NP, 16))] * 2,
        mesh=_mesh(),
        scratch_types=[
            pltpu.VMEM((4, CH), jnp.int32),
            pltpu.VMEM((4, CH), jnp.int32),
            pltpu.VMEM((512, 16), F32),
            pltpu.VMEM((512, 16), F32),
            pltpu.VMEM((512, 16), F32),
            pltpu.VMEM_SHARED((ACC_R, 16), F32),
            pltpu.SemaphoreType.DMA,
            pltpu.SemaphoreType.DMA,
        ],
        compiler_params=_SC_PARAMS,
    )
    def k(m_hbm, ds_hbm, zr_hbm, slo_hbm, shi_hbm,
          idxa_v, idxb_v, valsa_v, valsb_v, zw_v, acc, seml, sems):
        cid = lax.axis_index("c")
        sid = lax.axis_index("s")
        pltpu.sync_copy(zr_hbm, zw_v)
        z0 = sid * (NP // 16)
        for off, sz in _stripe_chunks(512):
            pltpu.sync_copy(zw_v.at[pl.ds(0, sz)],
                            acc.at[pl.ds(z0 + off, sz)])
        plsc.subcore_barrier()

        trows = ROWS // NS
        row0 = sid * trows

        def loads(r, idx_v, vals_v):
            pltpu.async_copy(ds_hbm.at[pl.ds(r, 4)], idx_v, seml)

            @pl.when(cid == 0)
            def _():
                pltpu.async_copy(
                    m_hbm.at[pl.ds(r * CH, 512), pl.ds(0, 16)], vals_v,
                    seml)

            @pl.when(cid == 1)
            def _():
                pltpu.async_copy(
                    m_hbm.at[pl.ds(r * CH, 512), pl.ds(16, 16)],
                    vals_v, seml)

        def drain_loads(r, idx_v, vals_v):
            pltpu.make_async_copy(ds_hbm.at[pl.ds(r, 4)], idx_v,
                                  seml).wait()
            pltpu.make_async_copy(
                m_hbm.at[pl.ds(r * CH, 512), pl.ds(0, 16)], vals_v,
                seml).wait()

        def scatters(idx_v, vals_v):
            return [pltpu.async_copy(vals_v.at[pl.ds(j * CH, CH)],
                                     acc.at[idx_v.at[j]], sems,
                                     add=True)
                    for j in range(4)]

        npairs = trows // 8
        loads(row0, idxa_v, valsa_v)

        def pair(i, carry):
            ra = row0 + (2 * i) * 4
            rb = row0 + (2 * i + 1) * 4
            loads(rb, idxb_v, valsb_v)
            drain_loads(ra, idxa_v, valsa_v)
            sa = scatters(idxa_v, valsa_v)
            drain_loads(rb, idxb_v, valsb_v)
            sb = scatters(idxb_v, valsb_v)
            for c in sa:
                c.wait()

            @pl.when(i < npairs - 1)
            def _():
                loads(row0 + (2 * i + 2) * 4, idxa_v, valsa_v)

            for c in sb:
                c.wait()
            return carry

        lax.fori_loop(0, npairs, pair, 0)
        plsc.subcore_barrier()

        w0 = sid * (NP // 16)
        for off, sz in _stripe_chunks(512):
            pltpu.sync_copy(acc.at[pl.ds(w0 + off, sz)],
                            zw_v.at[pl.ds(0, sz)])

            @pl.when(cid == 0)
            def _():
                pltpu.sync_copy(zw_v.at[pl.ds(0, sz)],
                                slo_hbm.at[pl.ds(w0 + off, sz)])

            @pl.when(cid == 1)
            def _():
                pltpu.sync_copy(zw_v.at[pl.ds(0, sz)],
                                shi_hbm.at[pl.ds(w0 + off, sz)])

    return k(m, dsts, zeros16)


def _sc_scatter_narrow(m, dsts, zeros8):
    @functools.partial(
        pl.kernel,
        out_type=_sds((2 * NP, 8)),
        mesh=_mesh(),
        scratch_types=[
            pltpu.VMEM((4, CH), jnp.int32),
            pltpu.VMEM((4, CH), jnp.int32),
            pltpu.VMEM((512, 8), F32),
            pltpu.VMEM((512, 8), F32),
            pltpu.VMEM((WCH, 8), F32),
            pltpu.VMEM_SHARED((ACC_R, 8), F32),
            pltpu.SemaphoreType.DMA,
            pltpu.SemaphoreType.DMA,
        ],
        compiler_params=_SC_PARAMS,
    )
    def k(m_hbm, ds_hbm, zr_hbm, out_hbm, idxa_v, idxb_v, valsa_v,
          valsb_v, zw_v, acc, seml, sems):
        cid = lax.axis_index("c")
        sid = lax.axis_index("s")
        pltpu.sync_copy(zr_hbm, zw_v)
        z0 = sid * (NP // 16)
        for off, sz in _stripe_chunks(WCH):
            pltpu.sync_copy(zw_v.at[pl.ds(0, sz)],
                            acc.at[pl.ds(z0 + off, sz)])
        plsc.subcore_barrier()

        row0 = cid * (ROWS // NC) + sid * (ROWS // NWK)

        def loads(r, idx_v, vals_v):
            pltpu.async_copy(ds_hbm.at[pl.ds(r, 4)], idx_v, seml)
            pltpu.async_copy(
                m_hbm.at[pl.ds(r * CH, 512), pl.ds(0, 8)], vals_v, seml)

        def drain_loads(r, idx_v, vals_v):
            pltpu.make_async_copy(ds_hbm.at[pl.ds(r, 4)], idx_v,
                                  seml).wait()
            pltpu.make_async_copy(
                m_hbm.at[pl.ds(r * CH, 512), pl.ds(0, 8)], vals_v,
                seml).wait()

        def scatters(idx_v, vals_v):
            return [pltpu.async_copy(vals_v.at[pl.ds(j * CH, CH)],
                                     acc.at[idx_v.at[j]], sems,
                                     add=True)
                    for j in range(4)]

        npairs = (ROWS // NWK) // 8
        loads(row0, idxa_v, valsa_v)

        def pair(i, carry):
            ra = row0 + (2 * i) * 4
            rb = row0 + (2 * i + 1) * 4
            loads(rb, idxb_v, valsb_v)
            drain_loads(ra, idxa_v, valsa_v)
            sa = scatters(idxa_v, valsa_v)
            drain_loads(rb, idxb_v, valsb_v)
            sb = scatters(idxb_v, valsb_v)
            for c in sa:
                c.wait()

            @pl.when(i < npairs - 1)
            def _():
                loads(row0 + (2 * i + 2) * 4, idxa_v, valsa_v)

            for c in sb:
                c.wait()
            return carry

        lax.fori_loop(0, npairs, pair, 0)
        plsc.subcore_barrier()

        w0 = sid * (NP // 16)
        for off, sz in _stripe_chunks(WCH):
            pltpu.sync_copy(acc.at[pl.ds(w0 + off, sz)],
                            zw_v.at[pl.ds(0, sz)])
            pltpu.sync_copy(zw_v.at[pl.ds(0, sz)],
                            out_hbm.at[pl.ds(cid * NP + w0 + off, sz)])

    return k(m, dsts, zeros8)


def _sc_count(dsts, cvals, zeros8):
    @functools.partial(
        pl.kernel,
        out_type=_sds((2 * NP, 8)),
        mesh=_mesh(),
        scratch_types=[
            pltpu.VMEM((GRP, CH), jnp.int32),
            pltpu.VMEM((CH, 8), F32),
            pltpu.VMEM((WCH, 8), F32),
            pltpu.VMEM_SHARED((ACC_R, 8), F32),
            pltpu.SemaphoreType.DMA,
            pltpu.SemaphoreType.DMA,
        ],
        compiler_params=_SC_PARAMS,
    )
    def k(ds_hbm, cv_hbm, zr_hbm, out_hbm, idx_v, vals_v, zw_v, acc,
          seml, sems):
        cid = lax.axis_index("c")
        sid = lax.axis_index("s")
        pltpu.sync_copy(zr_hbm, zw_v)
        z0 = sid * (NP // 16)
        for off, sz in _stripe_chunks(WCH):
            pltpu.sync_copy(zw_v.at[pl.ds(0, sz)],
                            acc.at[pl.ds(z0 + off, sz)])
        pltpu.sync_copy(cv_hbm, vals_v)
        plsc.subcore_barrier()

        row0 = cid * (ROWS // NC) + sid * (ROWS // NWK)

        def grp(g, carry):
            r = row0 + g * GRP
            ci = pltpu.async_copy(ds_hbm.at[pl.ds(r, GRP)], idx_v, seml)
            ci.wait()
            cps = [pltpu.async_copy(vals_v, acc.at[idx_v.at[j]], sems,
                                    add=True)
                   for j in range(GRP)]
            for c in cps:
                c.wait()
            return carry

        lax.fori_loop(0, (ROWS // NWK) // GRP, grp, 0)
        plsc.subcore_barrier()

        w0 = sid * (NP // 16)
        for off, sz in _stripe_chunks(WCH):
            pltpu.sync_copy(acc.at[pl.ds(w0 + off, sz)],
                            zw_v.at[pl.ds(0, sz)])
            pltpu.sync_copy(zw_v.at[pl.ds(0, sz)],
                            out_hbm.at[pl.ds(cid * NP + w0 + off, sz)])

    return k(dsts, cvals, zeros8)


def _prep_conv(p, fdim, fout):
    w0, w1, w2 = p["W"]
    b0, b1, b2 = p["b"]
    eye4 = jnp.eye(4, dtype=F32)
    w1a = jnp.zeros((32, 32), F32).at[:fdim].set(w0[:fdim])
    w1b = jnp.zeros((32, 32), F32).at[:fdim].set(w0[fdim:])
    if fout < 32:
        w2 = jnp.concatenate([w2, jnp.zeros((32, 32 - fout), F32)],
                             axis=1)
        b2 = jnp.concatenate([b2, jnp.zeros((32 - fout,), F32)])
    return {
        "w1k": jnp.concatenate(
            [jnp.kron(eye4, w1a), jnp.kron(eye4, w1b)], axis=0),
        "b1t": jnp.tile(b0.reshape(1, 32), (1, 4)),
        "bdw2": jnp.kron(eye4, w1),
        "b2t": jnp.tile(b1.reshape(1, 32), (1, 4)),
        "bdw3": jnp.kron(eye4, w2),
        "b3t": jnp.tile(b2.reshape(1, 32), (1, 4)),
    }


def kernel(x, edge_index, params):
    src = edge_index[0].astype(jnp.int32)
    dst = edge_index[1].astype(jnp.int32)
    npad = EPAD - NE
    pad_g = (jnp.arange(npad, dtype=jnp.int32) * 97) % NN
    pad_s = NN + (jnp.arange(npad, dtype=jnp.int32) % 8)
    dstg = jnp.concatenate([dst, pad_g]).reshape(ROWS, CH)
    srcg = jnp.concatenate([src, pad_g]).reshape(ROWS, CH)
    dsts = jnp.concatenate([dst, pad_s]).reshape(ROWS, CH)

    x128 = x.reshape(NN // 32, 128)
    x4 = x.reshape(NN // 4, 16)
    msel = (jnp.arange(128, dtype=jnp.int32)[:, None] % 4
            == jnp.arange(4, dtype=jnp.int32)[None, :]).astype(F32)
    bn2 = jnp.stack([params["bn"]["gamma"], params["bn"]["beta"]])
    zeros16 = jnp.zeros((512, 16), F32)
    zeros8 = jnp.zeros((WCH, 8), F32)
    cvals = (jnp.arange(8, dtype=jnp.int32)[None, :] == 0
             ).astype(F32) * jnp.ones((CH, 1), F32)

    lanes = jnp.arange(128)
    bc8 = jnp.kron(jnp.eye(16, dtype=F32),
                   jnp.zeros((8, 8), F32).at[0].set(1.0))
    d0 = jnp.zeros((128, 128), F32).at[(lanes // 16) * 8, lanes].set(1.0)
    d1 = jnp.zeros((128, 128), F32).at[64 + (lanes // 16) * 8,
                                       lanes].set(1.0)
    l64 = jnp.arange(64)
    sel = jnp.zeros((128, 64), F32).at[(l64 // 4) * 8 + l64 % 4,
                                       l64].set(1.0)
    exp16 = jnp.kron(jnp.eye(4, dtype=F32),
                     jnp.concatenate(
                         [jnp.eye(4, dtype=F32),
                          jnp.zeros((4, 28), F32)], axis=1))
    t8 = lanes // 16
    i16 = lanes % 16
    perm = jnp.zeros((256, 256), F32)
    perm = perm.at[lanes, t8 * 32 + i16].set(1.0)
    perm = perm.at[128 + lanes, t8 * 32 + 16 + i16].set(1.0)
    t16 = lanes // 8
    i8 = lanes % 8
    expn = jnp.zeros((128, 512), F32)
    expn = expn.at[lanes, t16 * 32 + i8].set(
        (i8 < 2).astype(F32))

    e1 = _prep_conv(params["enc1"], 4, 32)
    e2 = _prep_conv(params["enc2"], 32, 2)
    dc1 = _prep_conv(params["dec1"], 2, 32)
    dc2 = _prep_conv(params["dec2"], 32, 4)

    cnt2 = _sc_count(dsts, cvals, zeros8)
    dep = cnt2[:8]
    cp0 = cnt2[:NP].reshape(NP // 16, 128)
    cp1 = cnt2[NP:].reshape(NP // 16, 128)
    c16, c8w = _cnt_expand(cp0, cp1, bc8, d0, d1)
    c8 = c8w.reshape(NP // 8, 128)

    stats = _stats(x128)
    h = _state0(x4, stats, msel, bn2, exp16)

    def as_tab(t):
        return t.reshape(-1, 32)

    def g128(g):
        return g.reshape(EPAD // 4, 128)

    def conv(hpk, prep, last_relu):
        tab = as_tab(hpk)
        g1, g2 = _sc_gather(tab, tab, dstg, srcg, dep)
        return _mlp(last_relu, g128(g1), g128(g2), prep["w1k"],
                    prep["b1t"], prep["bdw2"], prep["b2t"],
                    prep["bdw3"], prep["b3t"])

    m = conv(h, e1, True)
    slo, shi = _sc_scatter_wide(m.reshape(EPAD, 32), dsts, zeros16)
    h = _state_wide(slo.reshape(NP // 8, 128),
                    shi.reshape(NP // 8, 128), c8, perm)
    m = conv(h, e2, True)
    s8 = _sc_scatter_narrow(m.reshape(EPAD, 32), dsts, zeros8)
    h = _state_narrow(s8[:NP].reshape(NP // 16, 128),
                      s8[NP:].reshape(NP // 16, 128), c16, expn)
    m = conv(h, dc1, True)
    slo, shi = _sc_scatter_wide(m.reshape(EPAD, 32), dsts, zeros16)
    h = _state_wide(slo.reshape(NP // 8, 128),
                    shi.reshape(NP // 8, 128), c8, perm)
    m = conv(h, dc2, False)
    s8 = _sc_scatter_narrow(m.reshape(EPAD, 32), dsts, zeros8)

    out = _final(s8[:NP].reshape(NP // 16, 128),
                 s8[NP:].reshape(NP // 16, 128), c16, sel)
    return out.reshape(NP, 4)[:NN]
